# Initial kernel scaffold; baseline (speedup 1.0000x reference)
import jax
import jax.numpy as jnp
from jax.experimental import pallas as pl


def kernel(node_features, edge_features, edge_indices, graph_indices, global_features, params):
    raise NotImplementedError("write your pallas kernel here")


# speedup vs baseline: 1.2358x; 1.2358x over previous
import functools

import jax
import jax.numpy as jnp
from jax.experimental import pallas as pl
from jax.experimental.pallas import tpu as pltpu

_INTERPRET = False

EPS = 1e-5
DH = 512
GPAD = 128

_f32 = jnp.float32


def _dot(a, b):
    return jnp.dot(a, b, preferred_element_type=_f32)


def _dott(a, b):
    return jax.lax.dot_general(a, b, (((0,), (0,)), ((), ())),
                               preferred_element_type=_f32)


def _sigmoid(y):
    return 1.0 / (1.0 + jnp.exp(-y))


def _afn_ab(s, musum, count, gam, bet, rm, rv):
    mu = musum * (1.0 / count)
    cov = s * (1.0 / count) - _dott(mu, mu)
    eye = (jax.lax.broadcasted_iota(jnp.int32, (DH, DH), 0)
           == jax.lax.broadcasted_iota(jnp.int32, (DH, DH), 1)).astype(_f32)
    d = cov * eye
    dr = jnp.sum(d, axis=1, keepdims=True)
    dc = jnp.sum(d, axis=0, keepdims=True)
    corr = jnp.clip(cov * jax.lax.rsqrt(dr * dc), -1.0, 1.0)
    mcol = jnp.sum(jnp.abs(corr), axis=0, keepdims=True) * (1.0 / DH)
    mx = jnp.max(mcol)
    w = jnp.exp(mcol - mx)
    w = w * (1.0 / jnp.sum(w))
    a = w * gam * jax.lax.rsqrt(rv + EPS)
    b = bet - rm * a
    return a, b


def _const_body(glob_ref, wde_ref, b1e_ref, wv2_ref, wo2_ref, bvf_ref, bo_ref,
                wbn_ref, wcn_ref, wdn_ref, b1n_ref,
                gge_ref, k2_ref, ggn_ref):
    glob = glob_ref[...]
    gge_ref[...] = _dot(glob, wde_ref[...]) + b1e_ref[...]
    m = _dot(wv2_ref[...], wo2_ref[...])
    k2_ref[...] = _dot(m, wbn_ref[...]) + wcn_ref[...]
    cvec = _dot(bvf_ref[...], wo2_ref[...]) + bo_ref[...]
    ggn_ref[...] = (_dot(glob, wdn_ref[...]) + b1n_ref[...]
                    + _dot(cvec, wbn_ref[...]))


def _pq_body(node_ref, gidx_ref, gge_ref, wa_ref, wb_ref, p_ref, q_ref):
    node = node_ref[...]
    g = gidx_ref[...]
    oh = (g == jax.lax.broadcasted_iota(jnp.int32, (1, 64), 1)).astype(_f32)
    p_ref[...] = _dot(node, wa_ref[...]) + _dot(oh, gge_ref[...])
    q_ref[...] = _dot(node, wb_ref[...])


def _pass1e_body(u_ref, eft_ref, wc_ref, h_ref, st_ref):
    i = pl.program_id(0)
    h = u_ref[...] + _dot(eft_ref[...], wc_ref[...])
    h_ref[...] = h

    @pl.when(i == 0)
    def _():
        st_ref[...] = jnp.zeros_like(st_ref)

    st_ref[0:DH, :] += _dott(h, h)
    st_ref[DH:DH + 1, :] += jnp.sum(h, axis=0, keepdims=True)


def _pass1n_body(node_ref, msg_ref, gidx_ref, k1_ref, k2_ref, ggn_ref,
                 h_ref, st_ref):
    i = pl.program_id(0)
    g = gidx_ref[...]
    oh = (g == jax.lax.broadcasted_iota(jnp.int32, (1, 64), 1)).astype(_f32)
    h = (_dot(node_ref[...], k1_ref[...]) + _dot(msg_ref[...], k2_ref[...])
         + _dot(oh, ggn_ref[...]))
    h_ref[...] = h

    @pl.when(i == 0)
    def _():
        st_ref[...] = jnp.zeros_like(st_ref)

    st_ref[0:DH, :] += _dott(h, h)
    st_ref[DH:DH + 1, :] += jnp.sum(h, axis=0, keepdims=True)


def _w_body(st_ref, gam_ref, bet_ref, rm_ref, rv_ref, ab_ref, *, count):
    a, b = _afn_ab(st_ref[0:DH, :], st_ref[DH:DH + 1, :], count,
                   gam_ref[...], bet_ref[...], rm_ref[...], rv_ref[...])
    ab_ref[0:1, :] = a
    ab_ref[1:2, :] = b


def _pass2e_body(h_ref, g_ref, ab_ref, w2_ref, b2_ref,
                 eo_ref, seg_ref, cnt_ref):
    i = pl.program_id(0)
    y = h_ref[...] * ab_ref[0:1, :] + ab_ref[1:2, :]
    y = y * _sigmoid(y)
    eo = _dot(y, w2_ref[...]) + b2_ref[...]
    eo_ref[...] = eo
    gf = g_ref[:, 0:1]
    ohf = (gf == jax.lax.broadcasted_iota(jnp.int32, (1, GPAD), 1)).astype(_f32)

    @pl.when(i == 0)
    def _():
        seg_ref[...] = jnp.zeros_like(seg_ref)
        cnt_ref[...] = jnp.zeros_like(cnt_ref)

    seg_ref[...] += _dott(ohf, eo)
    cnt_ref[0:1, :] += jnp.sum(ohf, axis=0, keepdims=True)


def _pass2n_body(h_ref, gidx_ref, ab_ref, w2_ref, b2_ref,
                 no_ref, seg_ref, cnt_ref):
    i = pl.program_id(0)
    y = h_ref[...] * ab_ref[0:1, :] + ab_ref[1:2, :]
    y = y * _sigmoid(y)
    no = _dot(y, w2_ref[...]) + b2_ref[...]
    no_ref[...] = no
    g = gidx_ref[...]
    oh = (g == jax.lax.broadcasted_iota(jnp.int32, (1, GPAD), 1)).astype(_f32)

    @pl.when(i == 0)
    def _():
        seg_ref[...] = jnp.zeros_like(seg_ref)
        cnt_ref[...] = jnp.zeros_like(cnt_ref)

    seg_ref[...] += _dott(oh, no)
    cnt_ref[0:1, :] += jnp.sum(oh, axis=0, keepdims=True)


def _glob_body(glb_ref, segn_ref, cntn_ref, sege_ref, cnte_ref,
               wg0_ref, wg1_ref, wg2_ref, b1g_ref,
               gam_ref, bet_ref, rm_ref, rv_ref, w2g_ref, b2g_ref,
               out_ref, *, gcount):
    g = glb_ref.shape[0]
    eye = (jax.lax.broadcasted_iota(jnp.int32, (g, GPAD), 0)
           == jax.lax.broadcasted_iota(jnp.int32, (g, GPAD), 1)).astype(_f32)
    nrec = 1.0 / jnp.maximum(cntn_ref[0:1, :], 1.0)
    erec = 1.0 / jnp.maximum(cnte_ref[0:1, :], 1.0)
    nmean = _dot(eye * nrec, segn_ref[...])
    emean = _dot(eye * erec, sege_ref[...])
    hg = (_dot(glb_ref[...], wg0_ref[...]) + _dot(nmean, wg1_ref[...])
          + _dot(emean, wg2_ref[...]) + b1g_ref[...])
    s = _dott(hg, hg)
    musum = jnp.sum(hg, axis=0, keepdims=True)
    a, b = _afn_ab(s, musum, gcount, gam_ref[...], bet_ref[...],
                   rm_ref[...], rv_ref[...])
    y = hg * a + b
    y = y * _sigmoid(y)
    out_ref[...] = _dot(y, w2g_ref[...]) + b2g_ref[...]


def _full(shape):
    return pl.BlockSpec(shape, lambda i: (0,) * len(shape))


def _afn2d(afn):
    return (afn['gamma'].reshape(1, DH), afn['beta'].reshape(1, DH),
            afn['rmean'].reshape(1, DH), afn['rvar'].reshape(1, DH))


def kernel(node_features, edge_features, edge_indices, graph_indices,
           global_features, params):
    N, U = node_features.shape
    E = edge_features.shape[0]
    G = global_features.shape[0]
    pe, pn, pg, pa = (params['edge_net'], params['node_net'],
                      params['global_net'], params['attn'])
    w1e, w1n, w1g = pe['W1'], pn['W1'], pg['W1']
    gidx2 = graph_indices.reshape(N, 1)

    gge, k2, ggn = pl.pallas_call(
        _const_body,
        out_shape=[jax.ShapeDtypeStruct((G, DH), _f32),
                   jax.ShapeDtypeStruct((U, DH), _f32),
                   jax.ShapeDtypeStruct((G, DH), _f32)],
        interpret=_INTERPRET,
    )(global_features, w1e[3 * U:4 * U], pe['b1'].reshape(1, DH),
      pa['Wv'].reshape(U, U), pa['Wo'].reshape(U, U),
      pa['bv'].reshape(1, U), pa['bo'].reshape(1, U),
      w1n[U:2 * U], w1n[2 * U:3 * U], w1n[3 * U:4 * U],
      pn['b1'].reshape(1, DH))

    BN = 2000
    nb_n = N // BN
    P, Q = pl.pallas_call(
        _pq_body,
        grid=(nb_n,),
        in_specs=[pl.BlockSpec((BN, U), lambda i: (i, 0)),
                  pl.BlockSpec((BN, 1), lambda i: (i, 0)),
                  _full((G, DH)), _full((U, DH)), _full((U, DH))],
        out_specs=[pl.BlockSpec((BN, DH), lambda i: (i, 0)),
                   pl.BlockSpec((BN, DH), lambda i: (i, 0))],
        out_shape=[jax.ShapeDtypeStruct((N, DH), _f32),
                   jax.ShapeDtypeStruct((N, DH), _f32)],
        interpret=_INTERPRET,
    )(node_features, gidx2, gge, w1e[0:U], w1e[U:2 * U])

    u = jnp.take(P, edge_indices[0], axis=0) + jnp.take(Q, edge_indices[1], axis=0)
    garr = jnp.broadcast_to(
        jnp.take(graph_indices, edge_indices[0])[:, None], (E, 16))

    BE = 2000
    nb_e = E // BE
    h_e, st_e = pl.pallas_call(
        _pass1e_body,
        grid=(nb_e,),
        in_specs=[pl.BlockSpec((BE, DH), lambda i: (i, 0)),
                  pl.BlockSpec((BE, U), lambda i: (i, 0)),
                  _full((U, DH))],
        out_specs=[pl.BlockSpec((BE, DH), lambda i: (i, 0)),
                   _full((DH + 1, DH))],
        out_shape=[jax.ShapeDtypeStruct((E, DH), _f32),
                   jax.ShapeDtypeStruct((DH + 1, DH), _f32)],
        interpret=_INTERPRET,
    )(u, edge_features, w1e[2 * U:3 * U])

    ab_e = pl.pallas_call(
        functools.partial(_w_body, count=float(E)),
        out_shape=jax.ShapeDtypeStruct((2, DH), _f32),
        interpret=_INTERPRET,
    )(st_e, *_afn2d(pe['afn']))

    edge_out, segE, cntE = pl.pallas_call(
        _pass2e_body,
        grid=(nb_e,),
        in_specs=[pl.BlockSpec((BE, DH), lambda i: (i, 0)),
                  pl.BlockSpec((BE, 16), lambda i: (i, 0)),
                  _full((2, DH)), _full((DH, U)), _full((1, U))],
        out_specs=[pl.BlockSpec((BE, U), lambda i: (i, 0)),
                   _full((GPAD, U)), _full((8, GPAD))],
        out_shape=[jax.ShapeDtypeStruct((E, U), _f32),
                   jax.ShapeDtypeStruct((GPAD, U), _f32),
                   jax.ShapeDtypeStruct((8, GPAD), _f32)],
        interpret=_INTERPRET,
    )(h_e, garr, ab_e, pe['W2'], pe['b2'].reshape(1, U))

    messages = jnp.zeros((N, U), _f32).at[edge_indices[1]].add(edge_out)

    h_n, st_n = pl.pallas_call(
        _pass1n_body,
        grid=(nb_n,),
        in_specs=[pl.BlockSpec((BN, U), lambda i: (i, 0)),
                  pl.BlockSpec((BN, U), lambda i: (i, 0)),
                  pl.BlockSpec((BN, 1), lambda i: (i, 0)),
                  _full((U, DH)), _full((U, DH)), _full((G, DH))],
        out_specs=[pl.BlockSpec((BN, DH), lambda i: (i, 0)),
                   _full((DH + 1, DH))],
        out_shape=[jax.ShapeDtypeStruct((N, DH), _f32),
                   jax.ShapeDtypeStruct((DH + 1, DH), _f32)],
        interpret=_INTERPRET,
    )(node_features, messages, gidx2, w1n[0:U], k2, ggn)

    ab_n = pl.pallas_call(
        functools.partial(_w_body, count=float(N)),
        out_shape=jax.ShapeDtypeStruct((2, DH), _f32),
        interpret=_INTERPRET,
    )(st_n, *_afn2d(pn['afn']))

    node_out, segN, cntN = pl.pallas_call(
        _pass2n_body,
        grid=(nb_n,),
        in_specs=[pl.BlockSpec((BN, DH), lambda i: (i, 0)),
                  pl.BlockSpec((BN, 1), lambda i: (i, 0)),
                  _full((2, DH)), _full((DH, U)), _full((1, U))],
        out_specs=[pl.BlockSpec((BN, U), lambda i: (i, 0)),
                   _full((GPAD, U)), _full((8, GPAD))],
        out_shape=[jax.ShapeDtypeStruct((N, U), _f32),
                   jax.ShapeDtypeStruct((GPAD, U), _f32),
                   jax.ShapeDtypeStruct((8, GPAD), _f32)],
        interpret=_INTERPRET,
    )(h_n, gidx2, ab_n, pn['W2'], pn['b2'].reshape(1, U))

    global_out = pl.pallas_call(
        functools.partial(_glob_body, gcount=float(G)),
        out_shape=jax.ShapeDtypeStruct((G, U), _f32),
        interpret=_INTERPRET,
    )(global_features, segN, cntN, segE, cntE,
      w1g[0:U], w1g[U:2 * U], w1g[2 * U:3 * U], pg['b1'].reshape(1, DH),
      *_afn2d(pg['afn']), pg['W2'], pg['b2'].reshape(1, U))

    return (node_out, edge_out, global_out)


# speedup vs baseline: 2.2113x; 1.7894x over previous
import functools

import jax
import jax.numpy as jnp
from jax import lax
from jax.experimental import pallas as pl
from jax.experimental.pallas import tpu as pltpu
from jax.experimental.pallas import tpu_sc as plsc

_INTERPRET = False

EPS = 1e-5
DH = 512
GPAD = 128
PW = DH + 128

_f32 = jnp.float32


def _dot(a, b):
    return jnp.dot(a, b, preferred_element_type=_f32)


def _dott(a, b):
    return jax.lax.dot_general(a, b, (((0,), (0,)), ((), ())),
                               preferred_element_type=_f32)


def _sigmoid(y):
    return 1.0 / (1.0 + jnp.exp(-y))


def _afn_ab(s, musum, count, gam, bet, rm, rv):
    mu = musum * (1.0 / count)
    cov = s * (1.0 / count) - _dott(mu, mu)
    eye = (jax.lax.broadcasted_iota(jnp.int32, (DH, DH), 0)
           == jax.lax.broadcasted_iota(jnp.int32, (DH, DH), 1)).astype(_f32)
    d = cov * eye
    dr = jnp.sum(d, axis=1, keepdims=True)
    dc = jnp.sum(d, axis=0, keepdims=True)
    corr = jnp.clip(cov * jax.lax.rsqrt(dr * dc), -1.0, 1.0)
    mcol = jnp.sum(jnp.abs(corr), axis=0, keepdims=True) * (1.0 / DH)
    mx = jnp.max(mcol)
    w = jnp.exp(mcol - mx)
    w = w * (1.0 / jnp.sum(w))
    a = w * gam * jax.lax.rsqrt(rv + EPS)
    b = bet - rm * a
    return a, b


_SC_MESH = dict(core_axis_name="c", subcore_axis_name="s")


def _make_gather(n, e, dh, pw):
    nw = 32
    epw = e // nw
    ch = 40
    nch = epw // ch

    mesh = plsc.VectorSubcoreMesh(**_SC_MESH)

    @functools.partial(
        pl.kernel, mesh=mesh,
        out_type=jax.ShapeDtypeStruct((e, pw), _f32),
        scratch_types=[pltpu.VMEM((nch, ch), jnp.int32),
                       pltpu.VMEM((nch, ch), jnp.int32),
                       pltpu.VMEM((ch, pw), _f32),
                       pltpu.VMEM((ch, dh), _f32),
                       pltpu.SemaphoreType.DMA,
                       pltpu.SemaphoreType.DMA],
    )
    def gather(p_hbm, q_hbm, src_hbm, dst_hbm, u_hbm,
               idx_s, idx_d, bufp, bufq, sem0, sem1):
        wid = lax.axis_index("s") * 2 + lax.axis_index("c")
        pltpu.sync_copy(src_hbm.at[wid], idx_s)
        pltpu.sync_copy(dst_hbm.at[wid], idx_d)

        def chunk(j, carry):
            base = pl.multiple_of(wid * epw + j * ch, 8)
            cp = pltpu.async_copy(p_hbm.at[idx_s.at[j]], bufp, sem0)
            cq = pltpu.async_copy(q_hbm.at[idx_d.at[j]], bufq, sem1)
            cp.wait()
            cq.wait()

            def row(r, c2):
                for l in range(dh // 16):
                    sl = pl.ds(l * 16, 16)
                    bufp[r, sl] = bufp[r, sl] + bufq[r, sl]
                return c2

            lax.fori_loop(0, ch, row, 0, unroll=False)
            pltpu.sync_copy(bufp, u_hbm.at[pl.ds(base, ch)])
            return carry

        lax.fori_loop(0, nch, chunk, 0, unroll=False)

    return gather


def _make_scatter(n, e, u):
    hc = u // 2
    ch = 128
    nchunks = e // ch
    base_tc = nchunks // 16
    extra = nchunks - base_tc * 16
    npt = 624

    mesh = plsc.VectorSubcoreMesh(**_SC_MESH)

    @functools.partial(
        pl.kernel, mesh=mesh,
        out_type=jax.ShapeDtypeStruct((n, u), _f32),
        scratch_types=[pltpu.VMEM((1, ch), jnp.int32),
                       pltpu.VMEM((ch, hc), _f32),
                       pltpu.VMEM_SHARED((n, hc), _f32)],
    )
    def scatter(eo_hbm, dst_hbm, zero_hbm, msg_hbm, idxr, ebuf, acc):
        c0 = pl.multiple_of(lax.axis_index("c") * hc, hc)
        t = lax.axis_index("s")
        r0 = pl.multiple_of(t * npt, 8)
        tail = n - 16 * npt
        pltpu.sync_copy(zero_hbm.at[pl.ds(0, npt)], acc.at[pl.ds(r0, npt)])

        @pl.when(t == 0)
        def _():
            pltpu.sync_copy(zero_hbm.at[pl.ds(0, tail)],
                            acc.at[pl.ds(16 * npt, tail)])

        plsc.subcore_barrier()

        tc = jnp.where(t < extra, base_tc + 1, base_tc)

        def chunk(k, carry):
            cid = t + k * 16
            e0 = pl.multiple_of(cid * ch, 8)
            pltpu.sync_copy(dst_hbm.at[cid], idxr)
            pltpu.sync_copy(eo_hbm.at[pl.ds(e0, ch), pl.ds(c0, hc)], ebuf)
            pltpu.sync_copy(ebuf, acc.at[idxr.at[0]], add=True)
            return carry

        lax.fori_loop(0, tc, chunk, 0, unroll=False)
        plsc.subcore_barrier()
        pltpu.sync_copy(acc.at[pl.ds(r0, npt)],
                        msg_hbm.at[pl.ds(r0, npt), pl.ds(c0, hc)])

        @pl.when(t == 0)
        def _():
            pltpu.sync_copy(acc.at[pl.ds(16 * npt, tail)],
                            msg_hbm.at[pl.ds(16 * npt, tail), pl.ds(c0, hc)])

    return scatter


def _const_body(glob_ref, wde_ref, b1e_ref, wv2_ref, wo2_ref, bvf_ref, bo_ref,
                wbn_ref, wcn_ref, wdn_ref, b1n_ref,
                gge_ref, k2_ref, ggn_ref):
    glob = glob_ref[...]
    gge_ref[...] = _dot(glob, wde_ref[...]) + b1e_ref[...]
    m = _dot(wv2_ref[...], wo2_ref[...])
    k2_ref[...] = _dot(m, wbn_ref[...]) + wcn_ref[...]
    cvec = _dot(bvf_ref[...], wo2_ref[...]) + bo_ref[...]
    ggn_ref[...] = (_dot(glob, wdn_ref[...]) + b1n_ref[...]
                    + _dot(cvec, wbn_ref[...]))


def _pq_body(node_ref, gidx_ref, gge_ref, wa_ref, wb_ref, p_ref, q_ref):
    node = node_ref[...]
    g = gidx_ref[...]
    oh = (g == jax.lax.broadcasted_iota(jnp.int32, (1, 64), 1)).astype(_f32)
    p_ref[:, 0:DH] = _dot(node, wa_ref[...]) + _dot(oh, gge_ref[...])
    p_ref[:, DH:PW] = jnp.broadcast_to(g.astype(_f32), (g.shape[0], PW - DH))
    q_ref[...] = _dot(node, wb_ref[...])


def _pass1e_body(u_ref, eft_ref, wc_ref, h_ref, st_ref):
    i = pl.program_id(0)
    h = u_ref[...] + _dot(eft_ref[...], wc_ref[...])
    h_ref[...] = h

    @pl.when(i == 0)
    def _():
        st_ref[...] = jnp.zeros_like(st_ref)

    st_ref[0:DH, :] += _dott(h, h)
    st_ref[DH:DH + 1, :] += jnp.sum(h, axis=0, keepdims=True)


def _pass1n_body(node_ref, msg_ref, gidx_ref, k1_ref, k2_ref, ggn_ref,
                 h_ref, st_ref):
    i = pl.program_id(0)
    g = gidx_ref[...]
    oh = (g == jax.lax.broadcasted_iota(jnp.int32, (1, 64), 1)).astype(_f32)
    h = (_dot(node_ref[...], k1_ref[...]) + _dot(msg_ref[...], k2_ref[...])
         + _dot(oh, ggn_ref[...]))
    h_ref[...] = h

    @pl.when(i == 0)
    def _():
        st_ref[...] = jnp.zeros_like(st_ref)

    st_ref[0:DH, :] += _dott(h, h)
    st_ref[DH:DH + 1, :] += jnp.sum(h, axis=0, keepdims=True)


def _w_body(st_ref, gam_ref, bet_ref, rm_ref, rv_ref, ab_ref, *, count):
    a, b = _afn_ab(st_ref[0:DH, :], st_ref[DH:DH + 1, :], count,
                   gam_ref[...], bet_ref[...], rm_ref[...], rv_ref[...])
    ab_ref[0:1, :] = a
    ab_ref[1:2, :] = b


def _pass2e_body(h_ref, g_ref, ab_ref, w2_ref, b2_ref,
                 eo_ref, seg_ref, cnt_ref):
    i = pl.program_id(0)
    y = h_ref[...] * ab_ref[0:1, :] + ab_ref[1:2, :]
    y = y * _sigmoid(y)
    eo = _dot(y, w2_ref[...]) + b2_ref[...]
    eo_ref[...] = eo
    gf = g_ref[:, 0:1]
    iot = jax.lax.broadcasted_iota(jnp.int32, (1, GPAD), 1).astype(_f32)
    ohf = (gf == iot).astype(_f32)

    @pl.when(i == 0)
    def _():
        seg_ref[...] = jnp.zeros_like(seg_ref)
        cnt_ref[...] = jnp.zeros_like(cnt_ref)

    seg_ref[...] += _dott(ohf, eo)
    cnt_ref[0:1, :] += jnp.sum(ohf, axis=0, keepdims=True)


def _pass2n_body(h_ref, gidx_ref, ab_ref, w2_ref, b2_ref,
                 no_ref, seg_ref, cnt_ref):
    i = pl.program_id(0)
    y = h_ref[...] * ab_ref[0:1, :] + ab_ref[1:2, :]
    y = y * _sigmoid(y)
    no = _dot(y, w2_ref[...]) + b2_ref[...]
    no_ref[...] = no
    g = gidx_ref[...]
    oh = (g == jax.lax.broadcasted_iota(jnp.int32, (1, GPAD), 1)).astype(_f32)

    @pl.when(i == 0)
    def _():
        seg_ref[...] = jnp.zeros_like(seg_ref)
        cnt_ref[...] = jnp.zeros_like(cnt_ref)

    seg_ref[...] += _dott(oh, no)
    cnt_ref[0:1, :] += jnp.sum(oh, axis=0, keepdims=True)


def _glob_body(glb_ref, segn_ref, cntn_ref, sege_ref, cnte_ref,
               wg0_ref, wg1_ref, wg2_ref, b1g_ref,
               gam_ref, bet_ref, rm_ref, rv_ref, w2g_ref, b2g_ref,
               out_ref, *, gcount):
    g = glb_ref.shape[0]
    eye = (jax.lax.broadcasted_iota(jnp.int32, (g, GPAD), 0)
           == jax.lax.broadcasted_iota(jnp.int32, (g, GPAD), 1)).astype(_f32)
    nrec = 1.0 / jnp.maximum(cntn_ref[0:1, :], 1.0)
    erec = 1.0 / jnp.maximum(cnte_ref[0:1, :], 1.0)
    nmean = _dot(eye * nrec, segn_ref[...])
    emean = _dot(eye * erec, sege_ref[...])
    hg = (_dot(glb_ref[...], wg0_ref[...]) + _dot(nmean, wg1_ref[...])
          + _dot(emean, wg2_ref[...]) + b1g_ref[...])
    s = _dott(hg, hg)
    musum = jnp.sum(hg, axis=0, keepdims=True)
    a, b = _afn_ab(s, musum, gcount, gam_ref[...], bet_ref[...],
                   rm_ref[...], rv_ref[...])
    y = hg * a + b
    y = y * _sigmoid(y)
    out_ref[...] = _dot(y, w2g_ref[...]) + b2g_ref[...]


def _full(shape):
    return pl.BlockSpec(shape, lambda i: (0,) * len(shape))


def _afn2d(afn):
    return (afn['gamma'].reshape(1, DH), afn['beta'].reshape(1, DH),
            afn['rmean'].reshape(1, DH), afn['rvar'].reshape(1, DH))


def kernel(node_features, edge_features, edge_indices, graph_indices,
           global_features, params):
    N, U = node_features.shape
    E = edge_features.shape[0]
    G = global_features.shape[0]
    pe, pn, pg, pa = (params['edge_net'], params['node_net'],
                      params['global_net'], params['attn'])
    w1e, w1n, w1g = pe['W1'], pn['W1'], pg['W1']
    gidx2 = graph_indices.reshape(N, 1)

    gge, k2, ggn = pl.pallas_call(
        _const_body,
        out_shape=[jax.ShapeDtypeStruct((G, DH), _f32),
                   jax.ShapeDtypeStruct((U, DH), _f32),
                   jax.ShapeDtypeStruct((G, DH), _f32)],
        interpret=_INTERPRET,
    )(global_features, w1e[3 * U:4 * U], pe['b1'].reshape(1, DH),
      pa['Wv'].reshape(U, U), pa['Wo'].reshape(U, U),
      pa['bv'].reshape(1, U), pa['bo'].reshape(1, U),
      w1n[U:2 * U], w1n[2 * U:3 * U], w1n[3 * U:4 * U],
      pn['b1'].reshape(1, DH))

    BN = 2000
    nb_n = N // BN
    P, Q = pl.pallas_call(
        _pq_body,
        grid=(nb_n,),
        in_specs=[pl.BlockSpec((BN, U), lambda i: (i, 0)),
                  pl.BlockSpec((BN, 1), lambda i: (i, 0)),
                  _full((G, DH)), _full((U, DH)), _full((U, DH))],
        out_specs=[pl.BlockSpec((BN, PW), lambda i: (i, 0)),
                   pl.BlockSpec((BN, DH), lambda i: (i, 0))],
        out_shape=[jax.ShapeDtypeStruct((N, PW), _f32),
                   jax.ShapeDtypeStruct((N, DH), _f32)],
        interpret=_INTERPRET,
    )(node_features, gidx2, gge, w1e[0:U], w1e[U:2 * U])

    src3 = edge_indices[0].reshape(32, E // (32 * 40), 40)
    dst3 = edge_indices[1].reshape(32, E // (32 * 40), 40)
    u = _make_gather(N, E, DH, PW)(P, Q, src3, dst3)

    BE = 2000
    nb_e = E // BE
    h_e, st_e = pl.pallas_call(
        _pass1e_body,
        grid=(nb_e,),
        in_specs=[pl.BlockSpec((BE, DH), lambda i: (i, 0)),
                  pl.BlockSpec((BE, U), lambda i: (i, 0)),
                  _full((U, DH))],
        out_specs=[pl.BlockSpec((BE, DH), lambda i: (i, 0)),
                   _full((DH + 1, DH))],
        out_shape=[jax.ShapeDtypeStruct((E, DH), _f32),
                   jax.ShapeDtypeStruct((DH + 1, DH), _f32)],
        interpret=_INTERPRET,
    )(u, edge_features, w1e[2 * U:3 * U])

    ab_e = pl.pallas_call(
        functools.partial(_w_body, count=float(E)),
        out_shape=jax.ShapeDtypeStruct((2, DH), _f32),
        interpret=_INTERPRET,
    )(st_e, *_afn2d(pe['afn']))

    edge_out, segE, cntE = pl.pallas_call(
        _pass2e_body,
        grid=(nb_e,),
        in_specs=[pl.BlockSpec((BE, DH), lambda i: (i, 0)),
                  pl.BlockSpec((BE, 128), lambda i: (i, DH // 128)),
                  _full((2, DH)), _full((DH, U)), _full((1, U))],
        out_specs=[pl.BlockSpec((BE, U), lambda i: (i, 0)),
                   _full((GPAD, U)), _full((8, GPAD))],
        out_shape=[jax.ShapeDtypeStruct((E, U), _f32),
                   jax.ShapeDtypeStruct((GPAD, U), _f32),
                   jax.ShapeDtypeStruct((8, GPAD), _f32)],
        interpret=_INTERPRET,
    )(h_e, u, ab_e, pe['W2'], pe['b2'].reshape(1, U))

    dst3_s = edge_indices[1].reshape(E // 128, 1, 128)
    zeros_h = jnp.zeros((624, U // 2), _f32)
    messages = _make_scatter(N, E, U)(edge_out, dst3_s, zeros_h)

    h_n, st_n = pl.pallas_call(
        _pass1n_body,
        grid=(nb_n,),
        in_specs=[pl.BlockSpec((BN, U), lambda i: (i, 0)),
                  pl.BlockSpec((BN, U), lambda i: (i, 0)),
                  pl.BlockSpec((BN, 1), lambda i: (i, 0)),
                  _full((U, DH)), _full((U, DH)), _full((G, DH))],
        out_specs=[pl.BlockSpec((BN, DH), lambda i: (i, 0)),
                   _full((DH + 1, DH))],
        out_shape=[jax.ShapeDtypeStruct((N, DH), _f32),
                   jax.ShapeDtypeStruct((DH + 1, DH), _f32)],
        interpret=_INTERPRET,
    )(node_features, messages, gidx2, w1n[0:U], k2, ggn)

    ab_n = pl.pallas_call(
        functools.partial(_w_body, count=float(N)),
        out_shape=jax.ShapeDtypeStruct((2, DH), _f32),
        interpret=_INTERPRET,
    )(st_n, *_afn2d(pn['afn']))

    node_out, segN, cntN = pl.pallas_call(
        _pass2n_body,
        grid=(nb_n,),
        in_specs=[pl.BlockSpec((BN, DH), lambda i: (i, 0)),
                  pl.BlockSpec((BN, 1), lambda i: (i, 0)),
                  _full((2, DH)), _full((DH, U)), _full((1, U))],
        out_specs=[pl.BlockSpec((BN, U), lambda i: (i, 0)),
                   _full((GPAD, U)), _full((8, GPAD))],
        out_shape=[jax.ShapeDtypeStruct((N, U), _f32),
                   jax.ShapeDtypeStruct((GPAD, U), _f32),
                   jax.ShapeDtypeStruct((8, GPAD), _f32)],
        interpret=_INTERPRET,
    )(h_n, gidx2, ab_n, pn['W2'], pn['b2'].reshape(1, U))

    global_out = pl.pallas_call(
        functools.partial(_glob_body, gcount=float(G)),
        out_shape=jax.ShapeDtypeStruct((G, U), _f32),
        interpret=_INTERPRET,
    )(global_features, segN, cntN, segE, cntE,
      w1g[0:U], w1g[U:2 * U], w1g[2 * U:3 * U], pg['b1'].reshape(1, DH),
      *_afn2d(pg['afn']), pg['W2'], pg['b2'].reshape(1, U))

    return (node_out, edge_out, global_out)


# speedup vs baseline: 2.6090x; 1.1799x over previous
import functools

import jax
import jax.numpy as jnp
from jax import lax
from jax.experimental import pallas as pl
from jax.experimental.pallas import tpu as pltpu
from jax.experimental.pallas import tpu_sc as plsc

_INTERPRET = False

EPS = 1e-5
DH = 512
GPAD = 128
PW = DH + 128

_f32 = jnp.float32


def _dot(a, b):
    return jnp.dot(a, b, preferred_element_type=_f32)


def _dott(a, b):
    return jax.lax.dot_general(a, b, (((0,), (0,)), ((), ())),
                               preferred_element_type=_f32)


def _sigmoid(y):
    return 1.0 / (1.0 + jnp.exp(-y))


def _afn_ab(s, musum, count, gam, bet, rm, rv):
    mu = musum * (1.0 / count)
    cov = s * (1.0 / count) - _dott(mu, mu)
    eye = (jax.lax.broadcasted_iota(jnp.int32, (DH, DH), 0)
           == jax.lax.broadcasted_iota(jnp.int32, (DH, DH), 1)).astype(_f32)
    d = cov * eye
    dr = jnp.sum(d, axis=1, keepdims=True)
    dc = jnp.sum(d, axis=0, keepdims=True)
    corr = jnp.clip(cov * jax.lax.rsqrt(dr * dc), -1.0, 1.0)
    mcol = jnp.sum(jnp.abs(corr), axis=0, keepdims=True) * (1.0 / DH)
    mx = jnp.max(mcol)
    w = jnp.exp(mcol - mx)
    w = w * (1.0 / jnp.sum(w))
    a = w * gam * jax.lax.rsqrt(rv + EPS)
    b = bet - rm * a
    return a, b


_SC_MESH = dict(core_axis_name="c", subcore_axis_name="s")


def _make_gather(n, e, dh, pw):
    nw = 32
    epw = e // nw
    ch = 40
    nch = epw // ch
    assert nch % 2 == 1

    mesh = plsc.VectorSubcoreMesh(**_SC_MESH)

    @functools.partial(
        pl.kernel, mesh=mesh,
        out_type=jax.ShapeDtypeStruct((e, pw), _f32),
        scratch_types=[pltpu.VMEM((nch, ch), jnp.int32),
                       pltpu.VMEM((nch, ch), jnp.int32),
                       pltpu.VMEM((ch, pw), _f32),
                       pltpu.VMEM((ch, dh), _f32),
                       pltpu.VMEM((ch, pw), _f32),
                       pltpu.VMEM((ch, dh), _f32),
                       pltpu.SemaphoreType.DMA,
                       pltpu.SemaphoreType.DMA,
                       pltpu.SemaphoreType.DMA,
                       pltpu.SemaphoreType.DMA],
    )
    def gather(p_hbm, q_hbm, src_hbm, dst_hbm, u_hbm,
               idx_s, idx_d, bufp0, bufq0, bufp1, bufq1, sp0, sq0, sp1, sq1):
        wid = lax.axis_index("s") * 2 + lax.axis_index("c")
        pltpu.sync_copy(src_hbm.at[wid], idx_s)
        pltpu.sync_copy(dst_hbm.at[wid], idx_d)

        def start(j, bp, bq, sp, sq):
            pltpu.async_copy(p_hbm.at[idx_s.at[j]], bp, sp)
            pltpu.async_copy(q_hbm.at[idx_d.at[j]], bq, sq)

        def proc(j, bp, bq, sp, sq):
            pltpu.make_async_copy(p_hbm.at[idx_s.at[j]], bp, sp).wait()
            pltpu.make_async_copy(q_hbm.at[idx_d.at[j]], bq, sq).wait()

            def row(r, c2):
                for l in range(dh // 16):
                    sl = pl.ds(l * 16, 16)
                    bp[r, sl] = bp[r, sl] + bq[r, sl]
                return c2

            lax.fori_loop(0, ch, row, 0, unroll=False)
            base = pl.multiple_of(wid * epw + j * ch, 8)
            pltpu.sync_copy(bp, u_hbm.at[pl.ds(base, ch)])

        start(0, bufp0, bufq0, sp0, sq0)

        def body(j, carry):
            @pl.when(j % 2 == 0)
            def _():
                start(j + 1, bufp1, bufq1, sp1, sq1)
                proc(j, bufp0, bufq0, sp0, sq0)

            @pl.when(j % 2 == 1)
            def _():
                start(j + 1, bufp0, bufq0, sp0, sq0)
                proc(j, bufp1, bufq1, sp1, sq1)

            return carry

        lax.fori_loop(0, nch - 1, body, 0, unroll=False)
        proc(nch - 1, bufp0, bufq0, sp0, sq0)

    return gather


def _make_scatter(n, e, u):
    hc = u // 2
    ch = 128
    nchunks = e // ch
    base_tc = nchunks // 16
    extra = nchunks - base_tc * 16
    npt = 624

    mesh = plsc.VectorSubcoreMesh(**_SC_MESH)

    @functools.partial(
        pl.kernel, mesh=mesh,
        out_type=jax.ShapeDtypeStruct((n, u), _f32),
        scratch_types=[pltpu.VMEM((1, ch), jnp.int32),
                       pltpu.VMEM((ch, hc), _f32),
                       pltpu.VMEM_SHARED((n, hc), _f32)],
    )
    def scatter(eo_hbm, dst_hbm, zero_hbm, msg_hbm, idxr, ebuf, acc):
        c0 = pl.multiple_of(lax.axis_index("c") * hc, hc)
        t = lax.axis_index("s")
        r0 = pl.multiple_of(t * npt, 8)
        tail = n - 16 * npt
        pltpu.sync_copy(zero_hbm.at[pl.ds(0, npt)], acc.at[pl.ds(r0, npt)])

        @pl.when(t == 0)
        def _():
            pltpu.sync_copy(zero_hbm.at[pl.ds(0, tail)],
                            acc.at[pl.ds(16 * npt, tail)])

        plsc.subcore_barrier()

        tc = jnp.where(t < extra, base_tc + 1, base_tc)

        def chunk(k, carry):
            cid = t + k * 16
            e0 = pl.multiple_of(cid * ch, 8)
            pltpu.sync_copy(dst_hbm.at[cid], idxr)
            pltpu.sync_copy(eo_hbm.at[pl.ds(e0, ch), pl.ds(c0, hc)], ebuf)
            pltpu.sync_copy(ebuf, acc.at[idxr.at[0]], add=True)
            return carry

        lax.fori_loop(0, tc, chunk, 0, unroll=False)
        plsc.subcore_barrier()
        pltpu.sync_copy(acc.at[pl.ds(r0, npt)],
                        msg_hbm.at[pl.ds(r0, npt), pl.ds(c0, hc)])

        @pl.when(t == 0)
        def _():
            pltpu.sync_copy(acc.at[pl.ds(16 * npt, tail)],
                            msg_hbm.at[pl.ds(16 * npt, tail), pl.ds(c0, hc)])

    return scatter


def _const_body(glob_ref, wde_ref, b1e_ref, wv2_ref, wo2_ref, bvf_ref, bo_ref,
                wbn_ref, wcn_ref, wdn_ref, b1n_ref,
                gge_ref, k2_ref, ggn_ref):
    glob = glob_ref[...]
    gge_ref[...] = _dot(glob, wde_ref[...]) + b1e_ref[...]
    m = _dot(wv2_ref[...], wo2_ref[...])
    k2_ref[...] = _dot(m, wbn_ref[...]) + wcn_ref[...]
    cvec = _dot(bvf_ref[...], wo2_ref[...]) + bo_ref[...]
    ggn_ref[...] = (_dot(glob, wdn_ref[...]) + b1n_ref[...]
                    + _dot(cvec, wbn_ref[...]))


def _pq_body(node_ref, gidx_ref, gge_ref, wa_ref, wb_ref, p_ref, q_ref):
    node = node_ref[...]
    g = gidx_ref[...]
    oh = (g == jax.lax.broadcasted_iota(jnp.int32, (1, 64), 1)).astype(_f32)
    p_ref[:, 0:DH] = _dot(node, wa_ref[...]) + _dot(oh, gge_ref[...])
    p_ref[:, DH:PW] = jnp.broadcast_to(g.astype(_f32), (g.shape[0], PW - DH))
    q_ref[...] = _dot(node, wb_ref[...])


def _pass1e_body(u_ref, eft_ref, wc_ref, h_ref, st_ref):
    i = pl.program_id(0)
    h = u_ref[...] + _dot(eft_ref[...], wc_ref[...])
    h_ref[...] = h

    @pl.when(i == 0)
    def _():
        st_ref[...] = jnp.zeros_like(st_ref)

    st_ref[0:DH, :] += _dott(h, h)
    st_ref[DH:DH + 1, :] += jnp.sum(h, axis=0, keepdims=True)


def _pass1n_body(node_ref, msg_ref, gidx_ref, k1_ref, k2_ref, ggn_ref,
                 h_ref, st_ref):
    i = pl.program_id(0)
    g = gidx_ref[...]
    oh = (g == jax.lax.broadcasted_iota(jnp.int32, (1, 64), 1)).astype(_f32)
    h = (_dot(node_ref[...], k1_ref[...]) + _dot(msg_ref[...], k2_ref[...])
         + _dot(oh, ggn_ref[...]))
    h_ref[...] = h

    @pl.when(i == 0)
    def _():
        st_ref[...] = jnp.zeros_like(st_ref)

    st_ref[0:DH, :] += _dott(h, h)
    st_ref[DH:DH + 1, :] += jnp.sum(h, axis=0, keepdims=True)


def _w_body(st_ref, gam_ref, bet_ref, rm_ref, rv_ref, ab_ref, *, count):
    a, b = _afn_ab(st_ref[0:DH, :], st_ref[DH:DH + 1, :], count,
                   gam_ref[...], bet_ref[...], rm_ref[...], rv_ref[...])
    ab_ref[0:1, :] = a
    ab_ref[1:2, :] = b


def _pass2e_body(h_ref, g_ref, ab_ref, w2_ref, b2_ref,
                 eo_ref, seg_ref, cnt_ref):
    i = pl.program_id(0)
    y = h_ref[...] * ab_ref[0:1, :] + ab_ref[1:2, :]
    y = y * _sigmoid(y)
    eo = _dot(y, w2_ref[...]) + b2_ref[...]
    eo_ref[...] = eo
    gf = g_ref[:, 0:1]
    iot = jax.lax.broadcasted_iota(jnp.int32, (1, GPAD), 1).astype(_f32)
    ohf = (gf == iot).astype(_f32)

    @pl.when(i == 0)
    def _():
        seg_ref[...] = jnp.zeros_like(seg_ref)
        cnt_ref[...] = jnp.zeros_like(cnt_ref)

    seg_ref[...] += _dott(ohf, eo)
    cnt_ref[0:1, :] += jnp.sum(ohf, axis=0, keepdims=True)


def _pass2n_body(h_ref, gidx_ref, ab_ref, w2_ref, b2_ref,
                 no_ref, seg_ref, cnt_ref):
    i = pl.program_id(0)
    y = h_ref[...] * ab_ref[0:1, :] + ab_ref[1:2, :]
    y = y * _sigmoid(y)
    no = _dot(y, w2_ref[...]) + b2_ref[...]
    no_ref[...] = no
    g = gidx_ref[...]
    oh = (g == jax.lax.broadcasted_iota(jnp.int32, (1, GPAD), 1)).astype(_f32)

    @pl.when(i == 0)
    def _():
        seg_ref[...] = jnp.zeros_like(seg_ref)
        cnt_ref[...] = jnp.zeros_like(cnt_ref)

    seg_ref[...] += _dott(oh, no)
    cnt_ref[0:1, :] += jnp.sum(oh, axis=0, keepdims=True)


def _glob_body(glb_ref, segn_ref, cntn_ref, sege_ref, cnte_ref,
               wg0_ref, wg1_ref, wg2_ref, b1g_ref,
               gam_ref, bet_ref, rm_ref, rv_ref, w2g_ref, b2g_ref,
               out_ref, *, gcount):
    g = glb_ref.shape[0]
    eye = (jax.lax.broadcasted_iota(jnp.int32, (g, GPAD), 0)
           == jax.lax.broadcasted_iota(jnp.int32, (g, GPAD), 1)).astype(_f32)
    nrec = 1.0 / jnp.maximum(cntn_ref[0:1, :], 1.0)
    erec = 1.0 / jnp.maximum(cnte_ref[0:1, :], 1.0)
    nmean = _dot(eye * nrec, segn_ref[...])
    emean = _dot(eye * erec, sege_ref[...])
    hg = (_dot(glb_ref[...], wg0_ref[...]) + _dot(nmean, wg1_ref[...])
          + _dot(emean, wg2_ref[...]) + b1g_ref[...])
    s = _dott(hg, hg)
    musum = jnp.sum(hg, axis=0, keepdims=True)
    a, b = _afn_ab(s, musum, gcount, gam_ref[...], bet_ref[...],
                   rm_ref[...], rv_ref[...])
    y = hg * a + b
    y = y * _sigmoid(y)
    out_ref[...] = _dot(y, w2g_ref[...]) + b2g_ref[...]


def _full(shape):
    return pl.BlockSpec(shape, lambda i: (0,) * len(shape))


def _afn2d(afn):
    return (afn['gamma'].reshape(1, DH), afn['beta'].reshape(1, DH),
            afn['rmean'].reshape(1, DH), afn['rvar'].reshape(1, DH))


def kernel(node_features, edge_features, edge_indices, graph_indices,
           global_features, params):
    N, U = node_features.shape
    E = edge_features.shape[0]
    G = global_features.shape[0]
    pe, pn, pg, pa = (params['edge_net'], params['node_net'],
                      params['global_net'], params['attn'])
    w1e, w1n, w1g = pe['W1'], pn['W1'], pg['W1']
    gidx2 = graph_indices.reshape(N, 1)

    gge, k2, ggn = pl.pallas_call(
        _const_body,
        out_shape=[jax.ShapeDtypeStruct((G, DH), _f32),
                   jax.ShapeDtypeStruct((U, DH), _f32),
                   jax.ShapeDtypeStruct((G, DH), _f32)],
        interpret=_INTERPRET,
    )(global_features, w1e[3 * U:4 * U], pe['b1'].reshape(1, DH),
      pa['Wv'].reshape(U, U), pa['Wo'].reshape(U, U),
      pa['bv'].reshape(1, U), pa['bo'].reshape(1, U),
      w1n[U:2 * U], w1n[2 * U:3 * U], w1n[3 * U:4 * U],
      pn['b1'].reshape(1, DH))

    BN = 2000
    nb_n = N // BN
    P, Q = pl.pallas_call(
        _pq_body,
        grid=(nb_n,),
        in_specs=[pl.BlockSpec((BN, U), lambda i: (i, 0)),
                  pl.BlockSpec((BN, 1), lambda i: (i, 0)),
                  _full((G, DH)), _full((U, DH)), _full((U, DH))],
        out_specs=[pl.BlockSpec((BN, PW), lambda i: (i, 0)),
                   pl.BlockSpec((BN, DH), lambda i: (i, 0))],
        out_shape=[jax.ShapeDtypeStruct((N, PW), _f32),
                   jax.ShapeDtypeStruct((N, DH), _f32)],
        interpret=_INTERPRET,
    )(node_features, gidx2, gge, w1e[0:U], w1e[U:2 * U])

    src3 = edge_indices[0].reshape(32, E // (32 * 40), 40)
    dst3 = edge_indices[1].reshape(32, E // (32 * 40), 40)
    u = _make_gather(N, E, DH, PW)(P, Q, src3, dst3)

    BE = 2000
    nb_e = E // BE
    h_e, st_e = pl.pallas_call(
        _pass1e_body,
        grid=(nb_e,),
        in_specs=[pl.BlockSpec((BE, DH), lambda i: (i, 0)),
                  pl.BlockSpec((BE, U), lambda i: (i, 0)),
                  _full((U, DH))],
        out_specs=[pl.BlockSpec((BE, DH), lambda i: (i, 0)),
                   _full((DH + 1, DH))],
        out_shape=[jax.ShapeDtypeStruct((E, DH), _f32),
                   jax.ShapeDtypeStruct((DH + 1, DH), _f32)],
        interpret=_INTERPRET,
    )(u, edge_features, w1e[2 * U:3 * U])

    ab_e = pl.pallas_call(
        functools.partial(_w_body, count=float(E)),
        out_shape=jax.ShapeDtypeStruct((2, DH), _f32),
        interpret=_INTERPRET,
    )(st_e, *_afn2d(pe['afn']))

    edge_out, segE, cntE = pl.pallas_call(
        _pass2e_body,
        grid=(nb_e,),
        in_specs=[pl.BlockSpec((BE, DH), lambda i: (i, 0)),
                  pl.BlockSpec((BE, 128), lambda i: (i, DH // 128)),
                  _full((2, DH)), _full((DH, U)), _full((1, U))],
        out_specs=[pl.BlockSpec((BE, U), lambda i: (i, 0)),
                   _full((GPAD, U)), _full((8, GPAD))],
        out_shape=[jax.ShapeDtypeStruct((E, U), _f32),
                   jax.ShapeDtypeStruct((GPAD, U), _f32),
                   jax.ShapeDtypeStruct((8, GPAD), _f32)],
        interpret=_INTERPRET,
    )(h_e, u, ab_e, pe['W2'], pe['b2'].reshape(1, U))

    dst3_s = edge_indices[1].reshape(E // 128, 1, 128)
    zeros_h = jnp.zeros((624, U // 2), _f32)
    messages = _make_scatter(N, E, U)(edge_out, dst3_s, zeros_h)

    h_n, st_n = pl.pallas_call(
        _pass1n_body,
        grid=(nb_n,),
        in_specs=[pl.BlockSpec((BN, U), lambda i: (i, 0)),
                  pl.BlockSpec((BN, U), lambda i: (i, 0)),
                  pl.BlockSpec((BN, 1), lambda i: (i, 0)),
                  _full((U, DH)), _full((U, DH)), _full((G, DH))],
        out_specs=[pl.BlockSpec((BN, DH), lambda i: (i, 0)),
                   _full((DH + 1, DH))],
        out_shape=[jax.ShapeDtypeStruct((N, DH), _f32),
                   jax.ShapeDtypeStruct((DH + 1, DH), _f32)],
        interpret=_INTERPRET,
    )(node_features, messages, gidx2, w1n[0:U], k2, ggn)

    ab_n = pl.pallas_call(
        functools.partial(_w_body, count=float(N)),
        out_shape=jax.ShapeDtypeStruct((2, DH), _f32),
        interpret=_INTERPRET,
    )(st_n, *_afn2d(pn['afn']))

    node_out, segN, cntN = pl.pallas_call(
        _pass2n_body,
        grid=(nb_n,),
        in_specs=[pl.BlockSpec((BN, DH), lambda i: (i, 0)),
                  pl.BlockSpec((BN, 1), lambda i: (i, 0)),
                  _full((2, DH)), _full((DH, U)), _full((1, U))],
        out_specs=[pl.BlockSpec((BN, U), lambda i: (i, 0)),
                   _full((GPAD, U)), _full((8, GPAD))],
        out_shape=[jax.ShapeDtypeStruct((N, U), _f32),
                   jax.ShapeDtypeStruct((GPAD, U), _f32),
                   jax.ShapeDtypeStruct((8, GPAD), _f32)],
        interpret=_INTERPRET,
    )(h_n, gidx2, ab_n, pn['W2'], pn['b2'].reshape(1, U))

    global_out = pl.pallas_call(
        functools.partial(_glob_body, gcount=float(G)),
        out_shape=jax.ShapeDtypeStruct((G, U), _f32),
        interpret=_INTERPRET,
    )(global_features, segN, cntN, segE, cntE,
      w1g[0:U], w1g[U:2 * U], w1g[2 * U:3 * U], pg['b1'].reshape(1, DH),
      *_afn2d(pg['afn']), pg['W2'], pg['b2'].reshape(1, U))

    return (node_out, edge_out, global_out)


# speedup vs baseline: 2.6909x; 1.0314x over previous
import functools

import jax
import jax.numpy as jnp
from jax import lax
from jax.experimental import pallas as pl
from jax.experimental.pallas import tpu as pltpu
from jax.experimental.pallas import tpu_sc as plsc

_INTERPRET = False

EPS = 1e-5
DH = 512
GPAD = 128
PW = DH + 128

_f32 = jnp.float32


def _dot(a, b):
    return jnp.dot(a, b, preferred_element_type=_f32)


def _dott(a, b):
    return jax.lax.dot_general(a, b, (((0,), (0,)), ((), ())),
                               preferred_element_type=_f32)


def _sigmoid(y):
    return 1.0 / (1.0 + jnp.exp(-y))


def _afn_ab(s, musum, count, gam, bet, rm, rv):
    mu = musum * (1.0 / count)
    cov = s * (1.0 / count) - _dott(mu, mu)
    eye = (jax.lax.broadcasted_iota(jnp.int32, (DH, DH), 0)
           == jax.lax.broadcasted_iota(jnp.int32, (DH, DH), 1)).astype(_f32)
    d = cov * eye
    dr = jnp.sum(d, axis=1, keepdims=True)
    dc = jnp.sum(d, axis=0, keepdims=True)
    corr = jnp.clip(cov * jax.lax.rsqrt(dr * dc), -1.0, 1.0)
    mcol = jnp.sum(jnp.abs(corr), axis=0, keepdims=True) * (1.0 / DH)
    mx = jnp.max(mcol)
    w = jnp.exp(mcol - mx)
    w = w * (1.0 / jnp.sum(w))
    a = w * gam * jax.lax.rsqrt(rv + EPS)
    b = bet - rm * a
    return a, b


_SC_MESH = dict(core_axis_name="c", subcore_axis_name="s")


def _make_gather(n, e, dh, pw):
    nw = 32
    epw = e // nw
    ch = 40
    nch = epw // ch
    assert nch % 2 == 1

    mesh = plsc.VectorSubcoreMesh(**_SC_MESH)

    @functools.partial(
        pl.kernel, mesh=mesh,
        out_type=jax.ShapeDtypeStruct((e, pw), _f32),
        scratch_types=[pltpu.VMEM((nch, ch), jnp.int32),
                       pltpu.VMEM((nch, ch), jnp.int32),
                       pltpu.VMEM((ch, pw), _f32),
                       pltpu.VMEM((ch, dh), _f32),
                       pltpu.VMEM((ch, pw), _f32),
                       pltpu.VMEM((ch, dh), _f32),
                       pltpu.SemaphoreType.DMA,
                       pltpu.SemaphoreType.DMA,
                       pltpu.SemaphoreType.DMA,
                       pltpu.SemaphoreType.DMA],
    )
    def gather(p_hbm, q_hbm, src_hbm, dst_hbm, u_hbm,
               idx_s, idx_d, bufp0, bufq0, bufp1, bufq1, sp0, sq0, sp1, sq1):
        wid = lax.axis_index("s") * 2 + lax.axis_index("c")
        pltpu.sync_copy(src_hbm.at[wid], idx_s)
        pltpu.sync_copy(dst_hbm.at[wid], idx_d)

        def start(j, bp, bq, sp, sq):
            pltpu.async_copy(p_hbm.at[idx_s.at[j]], bp, sp)
            pltpu.async_copy(q_hbm.at[idx_d.at[j]], bq, sq)

        def proc(j, bp, bq, sp, sq):
            pltpu.make_async_copy(p_hbm.at[idx_s.at[j]], bp, sp).wait()
            pltpu.make_async_copy(q_hbm.at[idx_d.at[j]], bq, sq).wait()

            def row(r, c2):
                for l in range(dh // 16):
                    sl = pl.ds(l * 16, 16)
                    bp[r, sl] = bp[r, sl] + bq[r, sl]
                return c2

            lax.fori_loop(0, ch, row, 0, unroll=False)
            base = pl.multiple_of(wid * epw + j * ch, 8)
            pltpu.sync_copy(bp, u_hbm.at[pl.ds(base, ch)])

        start(0, bufp0, bufq0, sp0, sq0)

        def body(j, carry):
            @pl.when(j % 2 == 0)
            def _():
                start(j + 1, bufp1, bufq1, sp1, sq1)
                proc(j, bufp0, bufq0, sp0, sq0)

            @pl.when(j % 2 == 1)
            def _():
                start(j + 1, bufp0, bufq0, sp0, sq0)
                proc(j, bufp1, bufq1, sp1, sq1)

            return carry

        lax.fori_loop(0, nch - 1, body, 0, unroll=False)
        proc(nch - 1, bufp0, bufq0, sp0, sq0)

    return gather


def _make_scatter(n, e, u):
    hc = u // 2
    ch = 128
    nchunks = e // ch
    base_tc = nchunks // 16
    extra = nchunks - base_tc * 16
    npt = 624

    mesh = plsc.VectorSubcoreMesh(**_SC_MESH)

    @functools.partial(
        pl.kernel, mesh=mesh,
        out_type=jax.ShapeDtypeStruct((n, u), _f32),
        scratch_types=[pltpu.VMEM((1, ch), jnp.int32),
                       pltpu.VMEM((ch, hc), _f32),
                       pltpu.VMEM_SHARED((n, hc), _f32)],
    )
    def scatter(eo_hbm, dst_hbm, zero_hbm, msg_hbm, idxr, ebuf, acc):
        c0 = pl.multiple_of(lax.axis_index("c") * hc, hc)
        t = lax.axis_index("s")
        r0 = pl.multiple_of(t * npt, 8)
        tail = n - 16 * npt
        pltpu.sync_copy(zero_hbm.at[pl.ds(0, npt)], acc.at[pl.ds(r0, npt)])

        @pl.when(t == 0)
        def _():
            pltpu.sync_copy(zero_hbm.at[pl.ds(0, tail)],
                            acc.at[pl.ds(16 * npt, tail)])

        plsc.subcore_barrier()

        tc = jnp.where(t < extra, base_tc + 1, base_tc)

        def chunk(k, carry):
            cid = t + k * 16
            e0 = pl.multiple_of(cid * ch, 8)
            pltpu.sync_copy(dst_hbm.at[cid], idxr)
            pltpu.sync_copy(eo_hbm.at[pl.ds(e0, ch), pl.ds(c0, hc)], ebuf)
            pltpu.sync_copy(ebuf, acc.at[idxr.at[0]], add=True)
            return carry

        lax.fori_loop(0, tc, chunk, 0, unroll=False)
        plsc.subcore_barrier()
        pltpu.sync_copy(acc.at[pl.ds(r0, npt)],
                        msg_hbm.at[pl.ds(r0, npt), pl.ds(c0, hc)])

        @pl.when(t == 0)
        def _():
            pltpu.sync_copy(acc.at[pl.ds(16 * npt, tail)],
                            msg_hbm.at[pl.ds(16 * npt, tail), pl.ds(c0, hc)])

    return scatter


def _const_body(glob_ref, wde_ref, b1e_ref, wv2_ref, wo2_ref, bvf_ref, bo_ref,
                wbn_ref, wcn_ref, wdn_ref, b1n_ref,
                gge_ref, k2_ref, ggn_ref):
    glob = glob_ref[...]
    gge_ref[...] = _dot(glob, wde_ref[...]) + b1e_ref[...]
    m = _dot(wv2_ref[...], wo2_ref[...])
    k2_ref[...] = _dot(m, wbn_ref[...]) + wcn_ref[...]
    cvec = _dot(bvf_ref[...], wo2_ref[...]) + bo_ref[...]
    ggn_ref[...] = (_dot(glob, wdn_ref[...]) + b1n_ref[...]
                    + _dot(cvec, wbn_ref[...]))


def _pq_body(node_ref, gidx_ref, gge_ref, wa_ref, wb_ref, p_ref, q_ref):
    node = node_ref[...]
    g = gidx_ref[...]
    oh = (g == jax.lax.broadcasted_iota(jnp.int32, (1, 64), 1)).astype(_f32)
    p_ref[:, 0:DH] = _dot(node, wa_ref[...]) + _dot(oh, gge_ref[...])
    p_ref[:, DH:PW] = jnp.broadcast_to(g.astype(_f32), (g.shape[0], PW - DH))
    q_ref[...] = _dot(node, wb_ref[...])


def _pass1e_body(u_ref, eft_ref, wc_ref, h_ref, st_ref):
    i = pl.program_id(0)
    h = u_ref[...] + _dot(eft_ref[...].astype(jnp.bfloat16), wc_ref[...])
    h_ref[...] = h.astype(jnp.bfloat16)

    @pl.when(i == 0)
    def _():
        st_ref[...] = jnp.zeros_like(st_ref)

    st_ref[0:DH, :] += _dott(h, h)
    st_ref[DH:DH + 1, :] += jnp.sum(h, axis=0, keepdims=True)


def _pass1n_body(node_ref, msg_ref, gidx_ref, k1_ref, k2_ref, ggn_ref,
                 h_ref, st_ref):
    i = pl.program_id(0)
    g = gidx_ref[...]
    oh = (g == jax.lax.broadcasted_iota(jnp.int32, (1, 64), 1)).astype(_f32)
    h = (_dot(node_ref[...], k1_ref[...]) + _dot(msg_ref[...], k2_ref[...])
         + _dot(oh, ggn_ref[...]))
    h_ref[...] = h

    @pl.when(i == 0)
    def _():
        st_ref[...] = jnp.zeros_like(st_ref)

    st_ref[0:DH, :] += _dott(h, h)
    st_ref[DH:DH + 1, :] += jnp.sum(h, axis=0, keepdims=True)


def _w_body(st_ref, gam_ref, bet_ref, rm_ref, rv_ref, ab_ref, *, count):
    a, b = _afn_ab(st_ref[0:DH, :], st_ref[DH:DH + 1, :], count,
                   gam_ref[...], bet_ref[...], rm_ref[...], rv_ref[...])
    ab_ref[0:1, :] = a
    ab_ref[1:2, :] = b


def _pass2e_body(h_ref, g_ref, ab_ref, w2_ref, b2_ref,
                 eo_ref, seg_ref, cnt_ref):
    i = pl.program_id(0)
    y = h_ref[...].astype(_f32) * ab_ref[0:1, :] + ab_ref[1:2, :]
    y = y * _sigmoid(y)
    eo = _dot(y.astype(jnp.bfloat16), w2_ref[...]) + b2_ref[...]
    eo_ref[...] = eo
    gf = g_ref[:, 0:1]
    iot = jax.lax.broadcasted_iota(jnp.int32, (1, GPAD), 1).astype(_f32)
    ohf = (gf == iot).astype(_f32)

    @pl.when(i == 0)
    def _():
        seg_ref[...] = jnp.zeros_like(seg_ref)
        cnt_ref[...] = jnp.zeros_like(cnt_ref)

    seg_ref[...] += _dott(ohf, eo)
    cnt_ref[0:1, :] += jnp.sum(ohf, axis=0, keepdims=True)


def _pass2n_body(h_ref, gidx_ref, ab_ref, w2_ref, b2_ref,
                 no_ref, seg_ref, cnt_ref):
    i = pl.program_id(0)
    y = h_ref[...] * ab_ref[0:1, :] + ab_ref[1:2, :]
    y = y * _sigmoid(y)
    no = _dot(y, w2_ref[...]) + b2_ref[...]
    no_ref[...] = no
    g = gidx_ref[...]
    oh = (g == jax.lax.broadcasted_iota(jnp.int32, (1, GPAD), 1)).astype(_f32)

    @pl.when(i == 0)
    def _():
        seg_ref[...] = jnp.zeros_like(seg_ref)
        cnt_ref[...] = jnp.zeros_like(cnt_ref)

    seg_ref[...] += _dott(oh, no)
    cnt_ref[0:1, :] += jnp.sum(oh, axis=0, keepdims=True)


def _glob_body(glb_ref, segn_ref, cntn_ref, sege_ref, cnte_ref,
               wg0_ref, wg1_ref, wg2_ref, b1g_ref,
               gam_ref, bet_ref, rm_ref, rv_ref, w2g_ref, b2g_ref,
               out_ref, *, gcount):
    g = glb_ref.shape[0]
    eye = (jax.lax.broadcasted_iota(jnp.int32, (g, GPAD), 0)
           == jax.lax.broadcasted_iota(jnp.int32, (g, GPAD), 1)).astype(_f32)
    nrec = 1.0 / jnp.maximum(cntn_ref[0:1, :], 1.0)
    erec = 1.0 / jnp.maximum(cnte_ref[0:1, :], 1.0)
    nmean = _dot(eye * nrec, segn_ref[...])
    emean = _dot(eye * erec, sege_ref[...])
    hg = (_dot(glb_ref[...], wg0_ref[...]) + _dot(nmean, wg1_ref[...])
          + _dot(emean, wg2_ref[...]) + b1g_ref[...])
    s = _dott(hg, hg)
    musum = jnp.sum(hg, axis=0, keepdims=True)
    a, b = _afn_ab(s, musum, gcount, gam_ref[...], bet_ref[...],
                   rm_ref[...], rv_ref[...])
    y = hg * a + b
    y = y * _sigmoid(y)
    out_ref[...] = _dot(y, w2g_ref[...]) + b2g_ref[...]


def _full(shape):
    return pl.BlockSpec(shape, lambda i: (0,) * len(shape))


def _afn2d(afn):
    return (afn['gamma'].reshape(1, DH), afn['beta'].reshape(1, DH),
            afn['rmean'].reshape(1, DH), afn['rvar'].reshape(1, DH))


def kernel(node_features, edge_features, edge_indices, graph_indices,
           global_features, params):
    N, U = node_features.shape
    E = edge_features.shape[0]
    G = global_features.shape[0]
    pe, pn, pg, pa = (params['edge_net'], params['node_net'],
                      params['global_net'], params['attn'])
    w1e, w1n, w1g = pe['W1'], pn['W1'], pg['W1']
    gidx2 = graph_indices.reshape(N, 1)

    gge, k2, ggn = pl.pallas_call(
        _const_body,
        out_shape=[jax.ShapeDtypeStruct((G, DH), _f32),
                   jax.ShapeDtypeStruct((U, DH), _f32),
                   jax.ShapeDtypeStruct((G, DH), _f32)],
        interpret=_INTERPRET,
    )(global_features, w1e[3 * U:4 * U], pe['b1'].reshape(1, DH),
      pa['Wv'].reshape(U, U), pa['Wo'].reshape(U, U),
      pa['bv'].reshape(1, U), pa['bo'].reshape(1, U),
      w1n[U:2 * U], w1n[2 * U:3 * U], w1n[3 * U:4 * U],
      pn['b1'].reshape(1, DH))

    BN = 2000
    nb_n = N // BN
    P, Q = pl.pallas_call(
        _pq_body,
        grid=(nb_n,),
        in_specs=[pl.BlockSpec((BN, U), lambda i: (i, 0)),
                  pl.BlockSpec((BN, 1), lambda i: (i, 0)),
                  _full((G, DH)), _full((U, DH)), _full((U, DH))],
        out_specs=[pl.BlockSpec((BN, PW), lambda i: (i, 0)),
                   pl.BlockSpec((BN, DH), lambda i: (i, 0))],
        out_shape=[jax.ShapeDtypeStruct((N, PW), _f32),
                   jax.ShapeDtypeStruct((N, DH), _f32)],
        interpret=_INTERPRET,
    )(node_features, gidx2, gge, w1e[0:U], w1e[U:2 * U])

    src3 = edge_indices[0].reshape(32, E // (32 * 40), 40)
    dst3 = edge_indices[1].reshape(32, E // (32 * 40), 40)
    u = _make_gather(N, E, DH, PW)(P, Q, src3, dst3)

    BE = 2000
    nb_e = E // BE
    h_e, st_e = pl.pallas_call(
        _pass1e_body,
        grid=(nb_e,),
        in_specs=[pl.BlockSpec((BE, DH), lambda i: (i, 0)),
                  pl.BlockSpec((BE, U), lambda i: (i, 0)),
                  _full((U, DH))],
        out_specs=[pl.BlockSpec((BE, DH), lambda i: (i, 0)),
                   _full((DH + 1, DH))],
        out_shape=[jax.ShapeDtypeStruct((E, DH), jnp.bfloat16),
                   jax.ShapeDtypeStruct((DH + 1, DH), _f32)],
        interpret=_INTERPRET,
    )(u, edge_features, w1e[2 * U:3 * U].astype(jnp.bfloat16))

    ab_e = pl.pallas_call(
        functools.partial(_w_body, count=float(E)),
        out_shape=jax.ShapeDtypeStruct((2, DH), _f32),
        interpret=_INTERPRET,
    )(st_e, *_afn2d(pe['afn']))

    edge_out, segE, cntE = pl.pallas_call(
        _pass2e_body,
        grid=(nb_e,),
        in_specs=[pl.BlockSpec((BE, DH), lambda i: (i, 0)),
                  pl.BlockSpec((BE, 128), lambda i: (i, DH // 128)),
                  _full((2, DH)), _full((DH, U)), _full((1, U))],
        out_specs=[pl.BlockSpec((BE, U), lambda i: (i, 0)),
                   _full((GPAD, U)), _full((8, GPAD))],
        out_shape=[jax.ShapeDtypeStruct((E, U), _f32),
                   jax.ShapeDtypeStruct((GPAD, U), _f32),
                   jax.ShapeDtypeStruct((8, GPAD), _f32)],
        interpret=_INTERPRET,
    )(h_e, u, ab_e, pe['W2'].astype(jnp.bfloat16), pe['b2'].reshape(1, U))

    dst3_s = edge_indices[1].reshape(E // 128, 1, 128)
    zeros_h = jnp.zeros((624, U // 2), _f32)
    messages = _make_scatter(N, E, U)(edge_out, dst3_s, zeros_h)

    h_n, st_n = pl.pallas_call(
        _pass1n_body,
        grid=(nb_n,),
        in_specs=[pl.BlockSpec((BN, U), lambda i: (i, 0)),
                  pl.BlockSpec((BN, U), lambda i: (i, 0)),
                  pl.BlockSpec((BN, 1), lambda i: (i, 0)),
                  _full((U, DH)), _full((U, DH)), _full((G, DH))],
        out_specs=[pl.BlockSpec((BN, DH), lambda i: (i, 0)),
                   _full((DH + 1, DH))],
        out_shape=[jax.ShapeDtypeStruct((N, DH), _f32),
                   jax.ShapeDtypeStruct((DH + 1, DH), _f32)],
        interpret=_INTERPRET,
    )(node_features, messages, gidx2, w1n[0:U], k2, ggn)

    ab_n = pl.pallas_call(
        functools.partial(_w_body, count=float(N)),
        out_shape=jax.ShapeDtypeStruct((2, DH), _f32),
        interpret=_INTERPRET,
    )(st_n, *_afn2d(pn['afn']))

    node_out, segN, cntN = pl.pallas_call(
        _pass2n_body,
        grid=(nb_n,),
        in_specs=[pl.BlockSpec((BN, DH), lambda i: (i, 0)),
                  pl.BlockSpec((BN, 1), lambda i: (i, 0)),
                  _full((2, DH)), _full((DH, U)), _full((1, U))],
        out_specs=[pl.BlockSpec((BN, U), lambda i: (i, 0)),
                   _full((GPAD, U)), _full((8, GPAD))],
        out_shape=[jax.ShapeDtypeStruct((N, U), _f32),
                   jax.ShapeDtypeStruct((GPAD, U), _f32),
                   jax.ShapeDtypeStruct((8, GPAD), _f32)],
        interpret=_INTERPRET,
    )(h_n, gidx2, ab_n, pn['W2'], pn['b2'].reshape(1, U))

    global_out = pl.pallas_call(
        functools.partial(_glob_body, gcount=float(G)),
        out_shape=jax.ShapeDtypeStruct((G, U), _f32),
        interpret=_INTERPRET,
    )(global_features, segN, cntN, segE, cntE,
      w1g[0:U], w1g[U:2 * U], w1g[2 * U:3 * U], pg['b1'].reshape(1, DH),
      *_afn2d(pg['afn']), pg['W2'], pg['b2'].reshape(1, U))

    return (node_out, edge_out, global_out)


# speedup vs baseline: 2.6961x; 1.0019x over previous
import functools

import jax
import jax.numpy as jnp
from jax import lax
from jax.experimental import pallas as pl
from jax.experimental.pallas import tpu as pltpu
from jax.experimental.pallas import tpu_sc as plsc

_INTERPRET = False

EPS = 1e-5
DH = 512
GPAD = 128
PW = DH + 128

_f32 = jnp.float32


def _dot(a, b):
    return jnp.dot(a, b, preferred_element_type=_f32)


def _dott(a, b):
    return jax.lax.dot_general(a, b, (((0,), (0,)), ((), ())),
                               preferred_element_type=_f32)


def _sigmoid(y):
    return 1.0 / (1.0 + jnp.exp(-y))


def _afn_ab(s, musum, count, gam, bet, rm, rv):
    mu = musum * (1.0 / count)
    cov = s * (1.0 / count) - _dott(mu, mu)
    eye = (jax.lax.broadcasted_iota(jnp.int32, (DH, DH), 0)
           == jax.lax.broadcasted_iota(jnp.int32, (DH, DH), 1)).astype(_f32)
    d = cov * eye
    dr = jnp.sum(d, axis=1, keepdims=True)
    dc = jnp.sum(d, axis=0, keepdims=True)
    corr = jnp.clip(cov * jax.lax.rsqrt(dr * dc), -1.0, 1.0)
    mcol = jnp.sum(jnp.abs(corr), axis=0, keepdims=True) * (1.0 / DH)
    mx = jnp.max(mcol)
    w = jnp.exp(mcol - mx)
    w = w * (1.0 / jnp.sum(w))
    a = w * gam * jax.lax.rsqrt(rv + EPS)
    b = bet - rm * a
    return a, b


_SC_MESH = dict(core_axis_name="c", subcore_axis_name="s")


def _make_gather(n, e, dh, pw):
    nw = 32
    epw = e // nw
    ch = 40
    nch = epw // ch
    assert nch % 2 == 1

    mesh = plsc.VectorSubcoreMesh(**_SC_MESH)

    @functools.partial(
        pl.kernel, mesh=mesh,
        out_type=jax.ShapeDtypeStruct((e, pw), _f32),
        scratch_types=[pltpu.VMEM((nch, ch), jnp.int32),
                       pltpu.VMEM((nch, ch), jnp.int32),
                       pltpu.VMEM((ch, pw), _f32),
                       pltpu.VMEM((ch, dh), _f32),
                       pltpu.VMEM((ch, pw), _f32),
                       pltpu.VMEM((ch, dh), _f32),
                       pltpu.SemaphoreType.DMA,
                       pltpu.SemaphoreType.DMA,
                       pltpu.SemaphoreType.DMA,
                       pltpu.SemaphoreType.DMA],
    )
    def gather(p_hbm, q_hbm, src_hbm, dst_hbm, u_hbm,
               idx_s, idx_d, bufp0, bufq0, bufp1, bufq1, sp0, sq0, sp1, sq1):
        wid = lax.axis_index("s") * 2 + lax.axis_index("c")
        pltpu.sync_copy(src_hbm.at[wid], idx_s)
        pltpu.sync_copy(dst_hbm.at[wid], idx_d)

        def start(j, bp, bq, sp, sq):
            pltpu.async_copy(p_hbm.at[idx_s.at[j]], bp, sp)
            pltpu.async_copy(q_hbm.at[idx_d.at[j]], bq, sq)

        def proc(j, bp, bq, sp, sq):
            pltpu.make_async_copy(p_hbm.at[idx_s.at[j]], bp, sp).wait()
            pltpu.make_async_copy(q_hbm.at[idx_d.at[j]], bq, sq).wait()

            def row(r, c2):
                for l in range(dh // 16):
                    sl = pl.ds(l * 16, 16)
                    bp[r, sl] = bp[r, sl] + bq[r, sl]
                return c2

            lax.fori_loop(0, ch, row, 0, unroll=False)
            base = pl.multiple_of(wid * epw + j * ch, 8)
            pltpu.sync_copy(bp, u_hbm.at[pl.ds(base, ch)])

        start(0, bufp0, bufq0, sp0, sq0)

        def body(j, carry):
            @pl.when(j % 2 == 0)
            def _():
                start(j + 1, bufp1, bufq1, sp1, sq1)
                proc(j, bufp0, bufq0, sp0, sq0)

            @pl.when(j % 2 == 1)
            def _():
                start(j + 1, bufp0, bufq0, sp0, sq0)
                proc(j, bufp1, bufq1, sp1, sq1)

            return carry

        lax.fori_loop(0, nch - 1, body, 0, unroll=False)
        proc(nch - 1, bufp0, bufq0, sp0, sq0)

    return gather


def _make_scatter(n, e, u):
    hc = u // 2
    ch = 128
    nchunks = e // ch
    base_tc = nchunks // 16
    extra = nchunks - base_tc * 16
    npt = 624

    mesh = plsc.VectorSubcoreMesh(**_SC_MESH)

    @functools.partial(
        pl.kernel, mesh=mesh,
        out_type=jax.ShapeDtypeStruct((n, u), _f32),
        scratch_types=[pltpu.VMEM((1, ch), jnp.int32),
                       pltpu.VMEM((ch, hc), _f32),
                       pltpu.VMEM_SHARED((n, hc), _f32)],
    )
    def scatter(eo_hbm, dst_hbm, zero_hbm, msg_hbm, idxr, ebuf, acc):
        c0 = pl.multiple_of(lax.axis_index("c") * hc, hc)
        t = lax.axis_index("s")
        r0 = pl.multiple_of(t * npt, 8)
        tail = n - 16 * npt
        pltpu.sync_copy(zero_hbm.at[pl.ds(0, npt)], acc.at[pl.ds(r0, npt)])

        @pl.when(t == 0)
        def _():
            pltpu.sync_copy(zero_hbm.at[pl.ds(0, tail)],
                            acc.at[pl.ds(16 * npt, tail)])

        plsc.subcore_barrier()

        tc = jnp.where(t < extra, base_tc + 1, base_tc)

        def chunk(k, carry):
            cid = t + k * 16
            e0 = pl.multiple_of(cid * ch, 8)
            pltpu.sync_copy(dst_hbm.at[cid], idxr)
            pltpu.sync_copy(eo_hbm.at[pl.ds(e0, ch), pl.ds(c0, hc)], ebuf)
            pltpu.sync_copy(ebuf, acc.at[idxr.at[0]], add=True)
            return carry

        lax.fori_loop(0, tc, chunk, 0, unroll=False)
        plsc.subcore_barrier()
        pltpu.sync_copy(acc.at[pl.ds(r0, npt)],
                        msg_hbm.at[pl.ds(r0, npt), pl.ds(c0, hc)])

        @pl.when(t == 0)
        def _():
            pltpu.sync_copy(acc.at[pl.ds(16 * npt, tail)],
                            msg_hbm.at[pl.ds(16 * npt, tail), pl.ds(c0, hc)])

    return scatter


def _const_body(glob_ref, wde_ref, b1e_ref, wv2_ref, wo2_ref, bvf_ref, bo_ref,
                wbn_ref, wcn_ref, wdn_ref, b1n_ref,
                gge_ref, k2_ref, ggn_ref):
    glob = glob_ref[...]
    gge_ref[...] = _dot(glob, wde_ref[...]) + b1e_ref[...]
    m = _dot(wv2_ref[...], wo2_ref[...])
    k2_ref[...] = _dot(m, wbn_ref[...]) + wcn_ref[...]
    cvec = _dot(bvf_ref[...], wo2_ref[...]) + bo_ref[...]
    ggn_ref[...] = (_dot(glob, wdn_ref[...]) + b1n_ref[...]
                    + _dot(cvec, wbn_ref[...]))


def _pq_body(node_ref, gidx_ref, gge_ref, wa_ref, wb_ref, p_ref, q_ref):
    node = node_ref[...]
    g = gidx_ref[...]
    oh = (g == jax.lax.broadcasted_iota(jnp.int32, (1, 64), 1)).astype(_f32)
    p_ref[:, 0:DH] = _dot(node, wa_ref[...]) + _dot(oh, gge_ref[...])
    p_ref[:, DH:PW] = jnp.broadcast_to(g.astype(_f32), (g.shape[0], PW - DH))
    q_ref[...] = _dot(node, wb_ref[...])


def _pass1e_body(u_ref, eft_ref, wc_ref, h_ref, st_ref):
    i = pl.program_id(0)
    h = u_ref[...] + _dot(eft_ref[...].astype(jnp.bfloat16), wc_ref[...])
    hb = h.astype(jnp.bfloat16)
    h_ref[...] = hb

    @pl.when(i == 0)
    def _():
        st_ref[...] = jnp.zeros_like(st_ref)

    st_ref[0:DH, :] += _dott(hb, hb)
    st_ref[DH:DH + 1, :] += jnp.sum(h, axis=0, keepdims=True)


def _pass1n_body(node_ref, msg_ref, gidx_ref, k1_ref, k2_ref, ggn_ref,
                 h_ref, st_ref):
    i = pl.program_id(0)
    g = gidx_ref[...]
    oh = (g == jax.lax.broadcasted_iota(jnp.int32, (1, 64), 1)).astype(_f32)
    h = (_dot(node_ref[...], k1_ref[...]) + _dot(msg_ref[...], k2_ref[...])
         + _dot(oh, ggn_ref[...]))
    h_ref[...] = h

    @pl.when(i == 0)
    def _():
        st_ref[...] = jnp.zeros_like(st_ref)

    st_ref[0:DH, :] += _dott(h, h)
    st_ref[DH:DH + 1, :] += jnp.sum(h, axis=0, keepdims=True)


def _w_body(st_ref, gam_ref, bet_ref, rm_ref, rv_ref, ab_ref, *, count):
    a, b = _afn_ab(st_ref[0:DH, :], st_ref[DH:DH + 1, :], count,
                   gam_ref[...], bet_ref[...], rm_ref[...], rv_ref[...])
    ab_ref[0:1, :] = a
    ab_ref[1:2, :] = b


def _pass2e_body(h_ref, g_ref, ab_ref, w2_ref, b2_ref,
                 eo_ref, seg_ref, cnt_ref):
    i = pl.program_id(0)
    y = h_ref[...].astype(_f32) * ab_ref[0:1, :] + ab_ref[1:2, :]
    y = y * _sigmoid(y)
    eo = _dot(y.astype(jnp.bfloat16), w2_ref[...]) + b2_ref[...]
    eo_ref[...] = eo
    gf = g_ref[:, 0:1]
    iot = jax.lax.broadcasted_iota(jnp.int32, (1, GPAD), 1).astype(_f32)
    ohf = (gf == iot).astype(_f32)

    @pl.when(i == 0)
    def _():
        seg_ref[...] = jnp.zeros_like(seg_ref)
        cnt_ref[...] = jnp.zeros_like(cnt_ref)

    seg_ref[...] += _dott(ohf, eo)
    cnt_ref[0:1, :] += jnp.sum(ohf, axis=0, keepdims=True)


def _pass2n_body(h_ref, gidx_ref, ab_ref, w2_ref, b2_ref,
                 no_ref, seg_ref, cnt_ref):
    i = pl.program_id(0)
    y = h_ref[...] * ab_ref[0:1, :] + ab_ref[1:2, :]
    y = y * _sigmoid(y)
    no = _dot(y, w2_ref[...]) + b2_ref[...]
    no_ref[...] = no
    g = gidx_ref[...]
    oh = (g == jax.lax.broadcasted_iota(jnp.int32, (1, GPAD), 1)).astype(_f32)

    @pl.when(i == 0)
    def _():
        seg_ref[...] = jnp.zeros_like(seg_ref)
        cnt_ref[...] = jnp.zeros_like(cnt_ref)

    seg_ref[...] += _dott(oh, no)
    cnt_ref[0:1, :] += jnp.sum(oh, axis=0, keepdims=True)


def _glob_body(glb_ref, segn_ref, cntn_ref, sege_ref, cnte_ref,
               wg0_ref, wg1_ref, wg2_ref, b1g_ref,
               gam_ref, bet_ref, rm_ref, rv_ref, w2g_ref, b2g_ref,
               out_ref, *, gcount):
    g = glb_ref.shape[0]
    eye = (jax.lax.broadcasted_iota(jnp.int32, (g, GPAD), 0)
           == jax.lax.broadcasted_iota(jnp.int32, (g, GPAD), 1)).astype(_f32)
    nrec = 1.0 / jnp.maximum(cntn_ref[0:1, :], 1.0)
    erec = 1.0 / jnp.maximum(cnte_ref[0:1, :], 1.0)
    nmean = _dot(eye * nrec, segn_ref[...])
    emean = _dot(eye * erec, sege_ref[...])
    hg = (_dot(glb_ref[...], wg0_ref[...]) + _dot(nmean, wg1_ref[...])
          + _dot(emean, wg2_ref[...]) + b1g_ref[...])
    s = _dott(hg, hg)
    musum = jnp.sum(hg, axis=0, keepdims=True)
    a, b = _afn_ab(s, musum, gcount, gam_ref[...], bet_ref[...],
                   rm_ref[...], rv_ref[...])
    y = hg * a + b
    y = y * _sigmoid(y)
    out_ref[...] = _dot(y, w2g_ref[...]) + b2g_ref[...]


def _full(shape):
    return pl.BlockSpec(shape, lambda i: (0,) * len(shape))


def _afn2d(afn):
    return (afn['gamma'].reshape(1, DH), afn['beta'].reshape(1, DH),
            afn['rmean'].reshape(1, DH), afn['rvar'].reshape(1, DH))


def kernel(node_features, edge_features, edge_indices, graph_indices,
           global_features, params):
    N, U = node_features.shape
    E = edge_features.shape[0]
    G = global_features.shape[0]
    pe, pn, pg, pa = (params['edge_net'], params['node_net'],
                      params['global_net'], params['attn'])
    w1e, w1n, w1g = pe['W1'], pn['W1'], pg['W1']
    gidx2 = graph_indices.reshape(N, 1)

    gge, k2, ggn = pl.pallas_call(
        _const_body,
        out_shape=[jax.ShapeDtypeStruct((G, DH), _f32),
                   jax.ShapeDtypeStruct((U, DH), _f32),
                   jax.ShapeDtypeStruct((G, DH), _f32)],
        interpret=_INTERPRET,
    )(global_features, w1e[3 * U:4 * U], pe['b1'].reshape(1, DH),
      pa['Wv'].reshape(U, U), pa['Wo'].reshape(U, U),
      pa['bv'].reshape(1, U), pa['bo'].reshape(1, U),
      w1n[U:2 * U], w1n[2 * U:3 * U], w1n[3 * U:4 * U],
      pn['b1'].reshape(1, DH))

    BN = 2000
    nb_n = N // BN
    P, Q = pl.pallas_call(
        _pq_body,
        grid=(nb_n,),
        in_specs=[pl.BlockSpec((BN, U), lambda i: (i, 0)),
                  pl.BlockSpec((BN, 1), lambda i: (i, 0)),
                  _full((G, DH)), _full((U, DH)), _full((U, DH))],
        out_specs=[pl.BlockSpec((BN, PW), lambda i: (i, 0)),
                   pl.BlockSpec((BN, DH), lambda i: (i, 0))],
        out_shape=[jax.ShapeDtypeStruct((N, PW), _f32),
                   jax.ShapeDtypeStruct((N, DH), _f32)],
        interpret=_INTERPRET,
    )(node_features, gidx2, gge, w1e[0:U], w1e[U:2 * U])

    src3 = edge_indices[0].reshape(32, E // (32 * 40), 40)
    dst3 = edge_indices[1].reshape(32, E // (32 * 40), 40)
    u = _make_gather(N, E, DH, PW)(P, Q, src3, dst3)

    BE = 2000
    nb_e = E // BE
    h_e, st_e = pl.pallas_call(
        _pass1e_body,
        grid=(nb_e,),
        in_specs=[pl.BlockSpec((BE, DH), lambda i: (i, 0)),
                  pl.BlockSpec((BE, U), lambda i: (i, 0)),
                  _full((U, DH))],
        out_specs=[pl.BlockSpec((BE, DH), lambda i: (i, 0)),
                   _full((DH + 1, DH))],
        out_shape=[jax.ShapeDtypeStruct((E, DH), jnp.bfloat16),
                   jax.ShapeDtypeStruct((DH + 1, DH), _f32)],
        interpret=_INTERPRET,
    )(u, edge_features, w1e[2 * U:3 * U].astype(jnp.bfloat16))

    ab_e = pl.pallas_call(
        functools.partial(_w_body, count=float(E)),
        out_shape=jax.ShapeDtypeStruct((2, DH), _f32),
        interpret=_INTERPRET,
    )(st_e, *_afn2d(pe['afn']))

    edge_out, segE, cntE = pl.pallas_call(
        _pass2e_body,
        grid=(nb_e,),
        in_specs=[pl.BlockSpec((BE, DH), lambda i: (i, 0)),
                  pl.BlockSpec((BE, 128), lambda i: (i, DH // 128)),
                  _full((2, DH)), _full((DH, U)), _full((1, U))],
        out_specs=[pl.BlockSpec((BE, U), lambda i: (i, 0)),
                   _full((GPAD, U)), _full((8, GPAD))],
        out_shape=[jax.ShapeDtypeStruct((E, U), _f32),
                   jax.ShapeDtypeStruct((GPAD, U), _f32),
                   jax.ShapeDtypeStruct((8, GPAD), _f32)],
        interpret=_INTERPRET,
    )(h_e, u, ab_e, pe['W2'].astype(jnp.bfloat16), pe['b2'].reshape(1, U))

    dst3_s = edge_indices[1].reshape(E // 128, 1, 128)
    zeros_h = jnp.zeros((624, U // 2), _f32)
    messages = _make_scatter(N, E, U)(edge_out, dst3_s, zeros_h)

    h_n, st_n = pl.pallas_call(
        _pass1n_body,
        grid=(nb_n,),
        in_specs=[pl.BlockSpec((BN, U), lambda i: (i, 0)),
                  pl.BlockSpec((BN, U), lambda i: (i, 0)),
                  pl.BlockSpec((BN, 1), lambda i: (i, 0)),
                  _full((U, DH)), _full((U, DH)), _full((G, DH))],
        out_specs=[pl.BlockSpec((BN, DH), lambda i: (i, 0)),
                   _full((DH + 1, DH))],
        out_shape=[jax.ShapeDtypeStruct((N, DH), _f32),
                   jax.ShapeDtypeStruct((DH + 1, DH), _f32)],
        interpret=_INTERPRET,
    )(node_features, messages, gidx2, w1n[0:U], k2, ggn)

    ab_n = pl.pallas_call(
        functools.partial(_w_body, count=float(N)),
        out_shape=jax.ShapeDtypeStruct((2, DH), _f32),
        interpret=_INTERPRET,
    )(st_n, *_afn2d(pn['afn']))

    node_out, segN, cntN = pl.pallas_call(
        _pass2n_body,
        grid=(nb_n,),
        in_specs=[pl.BlockSpec((BN, DH), lambda i: (i, 0)),
                  pl.BlockSpec((BN, 1), lambda i: (i, 0)),
                  _full((2, DH)), _full((DH, U)), _full((1, U))],
        out_specs=[pl.BlockSpec((BN, U), lambda i: (i, 0)),
                   _full((GPAD, U)), _full((8, GPAD))],
        out_shape=[jax.ShapeDtypeStruct((N, U), _f32),
                   jax.ShapeDtypeStruct((GPAD, U), _f32),
                   jax.ShapeDtypeStruct((8, GPAD), _f32)],
        interpret=_INTERPRET,
    )(h_n, gidx2, ab_n, pn['W2'], pn['b2'].reshape(1, U))

    global_out = pl.pallas_call(
        functools.partial(_glob_body, gcount=float(G)),
        out_shape=jax.ShapeDtypeStruct((G, U), _f32),
        interpret=_INTERPRET,
    )(global_features, segN, cntN, segE, cntE,
      w1g[0:U], w1g[U:2 * U], w1g[2 * U:3 * U], pg['b1'].reshape(1, DH),
      *_afn2d(pg['afn']), pg['W2'], pg['b2'].reshape(1, U))

    return (node_out, edge_out, global_out)


# speedup vs baseline: 2.8808x; 1.0685x over previous
import functools

import jax
import jax.numpy as jnp
from jax import lax
from jax.experimental import pallas as pl
from jax.experimental.pallas import tpu as pltpu
from jax.experimental.pallas import tpu_sc as plsc

_INTERPRET = False

EPS = 1e-5
DH = 512
GPAD = 128
PW = DH + 128

_f32 = jnp.float32


def _dot(a, b):
    return jnp.dot(a, b, preferred_element_type=_f32)


def _dott(a, b):
    return jax.lax.dot_general(a, b, (((0,), (0,)), ((), ())),
                               preferred_element_type=_f32)


def _sigmoid(y):
    return 1.0 / (1.0 + jnp.exp(-y))


def _afn_ab(s, musum, count, gam, bet, rm, rv):
    mu = musum * (1.0 / count)
    cov = s * (1.0 / count) - _dott(mu, mu)
    eye = (jax.lax.broadcasted_iota(jnp.int32, (DH, DH), 0)
           == jax.lax.broadcasted_iota(jnp.int32, (DH, DH), 1)).astype(_f32)
    d = cov * eye
    dr = jnp.sum(d, axis=1, keepdims=True)
    dc = jnp.sum(d, axis=0, keepdims=True)
    corr = jnp.clip(cov * jax.lax.rsqrt(dr * dc), -1.0, 1.0)
    mcol = jnp.sum(jnp.abs(corr), axis=0, keepdims=True) * (1.0 / DH)
    mx = jnp.max(mcol)
    w = jnp.exp(mcol - mx)
    w = w * (1.0 / jnp.sum(w))
    a = w * gam * jax.lax.rsqrt(rv + EPS)
    b = bet - rm * a
    return a, b


_SC_MESH = dict(core_axis_name="c", subcore_axis_name="s")


def _make_gather(n, e, dh, pw):
    nw = 32
    epw = e // nw
    ch = 40
    nch = epw // ch
    assert nch % 2 == 1

    mesh = plsc.VectorSubcoreMesh(**_SC_MESH)

    @functools.partial(
        pl.kernel, mesh=mesh,
        out_type=jax.ShapeDtypeStruct((e, pw), _f32),
        scratch_types=[pltpu.VMEM((nch, ch), jnp.int32),
                       pltpu.VMEM((nch, ch), jnp.int32),
                       pltpu.VMEM((ch, pw), _f32),
                       pltpu.VMEM((ch, dh), _f32),
                       pltpu.VMEM((ch, pw), _f32),
                       pltpu.VMEM((ch, dh), _f32),
                       pltpu.SemaphoreType.DMA,
                       pltpu.SemaphoreType.DMA,
                       pltpu.SemaphoreType.DMA,
                       pltpu.SemaphoreType.DMA],
    )
    def gather(p_hbm, q_hbm, src_hbm, dst_hbm, u_hbm,
               idx_s, idx_d, bufp0, bufq0, bufp1, bufq1, sp0, sq0, sp1, sq1):
        wid = lax.axis_index("s") * 2 + lax.axis_index("c")
        pltpu.sync_copy(src_hbm.at[wid], idx_s)
        pltpu.sync_copy(dst_hbm.at[wid], idx_d)

        def start(j, bp, bq, sp, sq):
            pltpu.async_copy(p_hbm.at[idx_s.at[j]], bp, sp)
            pltpu.async_copy(q_hbm.at[idx_d.at[j]], bq, sq)

        def proc(j, bp, bq, sp, sq):
            pltpu.make_async_copy(p_hbm.at[idx_s.at[j]], bp, sp).wait()
            pltpu.make_async_copy(q_hbm.at[idx_d.at[j]], bq, sq).wait()

            def row(r, c2):
                for l in range(dh // 16):
                    sl = pl.ds(l * 16, 16)
                    bp[r, sl] = bp[r, sl] + bq[r, sl]
                return c2

            lax.fori_loop(0, ch, row, 0, unroll=False)
            base = pl.multiple_of(wid * epw + j * ch, 8)
            pltpu.sync_copy(bp, u_hbm.at[pl.ds(base, ch)])

        start(0, bufp0, bufq0, sp0, sq0)

        def body(j, carry):
            @pl.when(j % 2 == 0)
            def _():
                start(j + 1, bufp1, bufq1, sp1, sq1)
                proc(j, bufp0, bufq0, sp0, sq0)

            @pl.when(j % 2 == 1)
            def _():
                start(j + 1, bufp0, bufq0, sp0, sq0)
                proc(j, bufp1, bufq1, sp1, sq1)

            return carry

        lax.fori_loop(0, nch - 1, body, 0, unroll=False)
        proc(nch - 1, bufp0, bufq0, sp0, sq0)

    return gather


def _make_scatter(n, e, u):
    hc = u // 2
    ch = 128
    nchunks = e // ch
    base_tc = nchunks // 16
    extra = nchunks - base_tc * 16
    npt = 624

    mesh = plsc.VectorSubcoreMesh(**_SC_MESH)

    assert base_tc % 2 == 0

    @functools.partial(
        pl.kernel, mesh=mesh,
        out_type=jax.ShapeDtypeStruct((n, u), _f32),
        scratch_types=[pltpu.VMEM((1, ch), jnp.int32),
                       pltpu.VMEM((ch, hc), _f32),
                       pltpu.VMEM((1, ch), jnp.int32),
                       pltpu.VMEM((ch, hc), _f32),
                       pltpu.SemaphoreType.DMA,
                       pltpu.SemaphoreType.DMA,
                       pltpu.SemaphoreType.DMA,
                       pltpu.SemaphoreType.DMA,
                       pltpu.VMEM_SHARED((n, hc), _f32)],
    )
    def scatter(eo_hbm, dst_hbm, zero_hbm, msg_hbm,
                idx0, eb0, idx1, eb1, si0, se0, si1, se1, acc):
        c0 = pl.multiple_of(lax.axis_index("c") * hc, hc)
        t = lax.axis_index("s")
        r0 = pl.multiple_of(t * npt, 8)
        tail = n - 16 * npt
        pltpu.sync_copy(zero_hbm.at[pl.ds(0, npt)], acc.at[pl.ds(r0, npt)])

        @pl.when(t == 0)
        def _():
            pltpu.sync_copy(zero_hbm.at[pl.ds(0, tail)],
                            acc.at[pl.ds(16 * npt, tail)])

        plsc.subcore_barrier()

        def start(k, bi, be, si, se):
            cid = t + k * 16
            e0 = pl.multiple_of(cid * ch, 8)
            pltpu.async_copy(dst_hbm.at[cid], bi, si)
            pltpu.async_copy(eo_hbm.at[pl.ds(e0, ch), pl.ds(c0, hc)], be, se)

        def proc(k, bi, be, si, se):
            cid = t + k * 16
            e0 = pl.multiple_of(cid * ch, 8)
            pltpu.make_async_copy(dst_hbm.at[cid], bi, si).wait()
            pltpu.make_async_copy(
                eo_hbm.at[pl.ds(e0, ch), pl.ds(c0, hc)], be, se).wait()
            pltpu.sync_copy(be, acc.at[bi.at[0]], add=True)

        start(0, idx0, eb0, si0, se0)

        def body(k, carry):
            @pl.when(k % 2 == 0)
            def _():
                start(k + 1, idx1, eb1, si1, se1)
                proc(k, idx0, eb0, si0, se0)

            @pl.when(k % 2 == 1)
            def _():
                start(k + 1, idx0, eb0, si0, se0)
                proc(k, idx1, eb1, si1, se1)

            return carry

        lax.fori_loop(0, base_tc - 1, body, 0, unroll=False)
        proc(base_tc - 1, idx1, eb1, si1, se1)

        @pl.when(t < extra)
        def _():
            start(base_tc, idx0, eb0, si0, se0)
            proc(base_tc, idx0, eb0, si0, se0)

        plsc.subcore_barrier()
        pltpu.sync_copy(acc.at[pl.ds(r0, npt)],
                        msg_hbm.at[pl.ds(r0, npt), pl.ds(c0, hc)])

        @pl.when(t == 0)
        def _():
            pltpu.sync_copy(acc.at[pl.ds(16 * npt, tail)],
                            msg_hbm.at[pl.ds(16 * npt, tail), pl.ds(c0, hc)])

    return scatter


def _const_body(glob_ref, wde_ref, b1e_ref, wv2_ref, wo2_ref, bvf_ref, bo_ref,
                wbn_ref, wcn_ref, wdn_ref, b1n_ref,
                gge_ref, k2_ref, ggn_ref):
    glob = glob_ref[...]
    gge_ref[...] = _dot(glob, wde_ref[...]) + b1e_ref[...]
    m = _dot(wv2_ref[...], wo2_ref[...])
    k2_ref[...] = _dot(m, wbn_ref[...]) + wcn_ref[...]
    cvec = _dot(bvf_ref[...], wo2_ref[...]) + bo_ref[...]
    ggn_ref[...] = (_dot(glob, wdn_ref[...]) + b1n_ref[...]
                    + _dot(cvec, wbn_ref[...]))


def _pq_body(node_ref, gidx_ref, gge_ref, wa_ref, wb_ref, p_ref, q_ref):
    node = node_ref[...]
    g = gidx_ref[...]
    oh = (g == jax.lax.broadcasted_iota(jnp.int32, (1, 64), 1)).astype(_f32)
    p_ref[:, 0:DH] = _dot(node, wa_ref[...]) + _dot(oh, gge_ref[...])
    p_ref[:, DH:PW] = jnp.broadcast_to(g.astype(_f32), (g.shape[0], PW - DH))
    q_ref[...] = _dot(node, wb_ref[...])


def _pass1e_body(u_ref, eft_ref, wc_ref, h_ref, st_ref):
    i = pl.program_id(0)
    h = u_ref[...] + _dot(eft_ref[...].astype(jnp.bfloat16), wc_ref[...])
    hb = h.astype(jnp.bfloat16)
    h_ref[...] = hb

    @pl.when(i == 0)
    def _():
        st_ref[...] = jnp.zeros_like(st_ref)

    st_ref[0:DH, :] += _dott(hb, hb)
    st_ref[DH:DH + 1, :] += jnp.sum(h, axis=0, keepdims=True)


def _pass1n_body(node_ref, msg_ref, gidx_ref, k1_ref, k2_ref, ggn_ref,
                 h_ref, st_ref):
    i = pl.program_id(0)
    g = gidx_ref[...]
    oh = (g == jax.lax.broadcasted_iota(jnp.int32, (1, 64), 1)).astype(_f32)
    h = (_dot(node_ref[...], k1_ref[...]) + _dot(msg_ref[...], k2_ref[...])
         + _dot(oh, ggn_ref[...]))
    h_ref[...] = h

    @pl.when(i == 0)
    def _():
        st_ref[...] = jnp.zeros_like(st_ref)

    st_ref[0:DH, :] += _dott(h, h)
    st_ref[DH:DH + 1, :] += jnp.sum(h, axis=0, keepdims=True)


def _w_body(st_ref, gam_ref, bet_ref, rm_ref, rv_ref, ab_ref, *, count):
    a, b = _afn_ab(st_ref[0:DH, :], st_ref[DH:DH + 1, :], count,
                   gam_ref[...], bet_ref[...], rm_ref[...], rv_ref[...])
    ab_ref[0:1, :] = a
    ab_ref[1:2, :] = b


def _pass2e_body(h_ref, g_ref, ab_ref, w2_ref, b2_ref,
                 eo_ref, seg_ref, cnt_ref):
    i = pl.program_id(0)
    y = h_ref[...].astype(_f32) * ab_ref[0:1, :] + ab_ref[1:2, :]
    y = y * _sigmoid(y)
    eo = _dot(y.astype(jnp.bfloat16), w2_ref[...]) + b2_ref[...]
    eo_ref[...] = eo
    gf = g_ref[:, 0:1]
    iot = jax.lax.broadcasted_iota(jnp.int32, (1, GPAD), 1).astype(_f32)
    ohf = (gf == iot).astype(_f32)

    @pl.when(i == 0)
    def _():
        seg_ref[...] = jnp.zeros_like(seg_ref)
        cnt_ref[...] = jnp.zeros_like(cnt_ref)

    seg_ref[...] += _dott(ohf, eo)
    cnt_ref[0:1, :] += jnp.sum(ohf, axis=0, keepdims=True)


def _pass2n_body(h_ref, gidx_ref, ab_ref, w2_ref, b2_ref,
                 no_ref, seg_ref, cnt_ref):
    i = pl.program_id(0)
    y = h_ref[...] * ab_ref[0:1, :] + ab_ref[1:2, :]
    y = y * _sigmoid(y)
    no = _dot(y, w2_ref[...]) + b2_ref[...]
    no_ref[...] = no
    g = gidx_ref[...]
    oh = (g == jax.lax.broadcasted_iota(jnp.int32, (1, GPAD), 1)).astype(_f32)

    @pl.when(i == 0)
    def _():
        seg_ref[...] = jnp.zeros_like(seg_ref)
        cnt_ref[...] = jnp.zeros_like(cnt_ref)

    seg_ref[...] += _dott(oh, no)
    cnt_ref[0:1, :] += jnp.sum(oh, axis=0, keepdims=True)


def _glob_body(glb_ref, segn_ref, cntn_ref, sege_ref, cnte_ref,
               wg0_ref, wg1_ref, wg2_ref, b1g_ref,
               gam_ref, bet_ref, rm_ref, rv_ref, w2g_ref, b2g_ref,
               out_ref, *, gcount):
    g = glb_ref.shape[0]
    eye = (jax.lax.broadcasted_iota(jnp.int32, (g, GPAD), 0)
           == jax.lax.broadcasted_iota(jnp.int32, (g, GPAD), 1)).astype(_f32)
    nrec = 1.0 / jnp.maximum(cntn_ref[0:1, :], 1.0)
    erec = 1.0 / jnp.maximum(cnte_ref[0:1, :], 1.0)
    nmean = _dot(eye * nrec, segn_ref[...])
    emean = _dot(eye * erec, sege_ref[...])
    hg = (_dot(glb_ref[...], wg0_ref[...]) + _dot(nmean, wg1_ref[...])
          + _dot(emean, wg2_ref[...]) + b1g_ref[...])
    s = _dott(hg, hg)
    musum = jnp.sum(hg, axis=0, keepdims=True)
    a, b = _afn_ab(s, musum, gcount, gam_ref[...], bet_ref[...],
                   rm_ref[...], rv_ref[...])
    y = hg * a + b
    y = y * _sigmoid(y)
    out_ref[...] = _dot(y, w2g_ref[...]) + b2g_ref[...]


def _full(shape):
    return pl.BlockSpec(shape, lambda i: (0,) * len(shape))


def _afn2d(afn):
    return (afn['gamma'].reshape(1, DH), afn['beta'].reshape(1, DH),
            afn['rmean'].reshape(1, DH), afn['rvar'].reshape(1, DH))


def kernel(node_features, edge_features, edge_indices, graph_indices,
           global_features, params):
    N, U = node_features.shape
    E = edge_features.shape[0]
    G = global_features.shape[0]
    pe, pn, pg, pa = (params['edge_net'], params['node_net'],
                      params['global_net'], params['attn'])
    w1e, w1n, w1g = pe['W1'], pn['W1'], pg['W1']
    gidx2 = graph_indices.reshape(N, 1)

    gge, k2, ggn = pl.pallas_call(
        _const_body,
        out_shape=[jax.ShapeDtypeStruct((G, DH), _f32),
                   jax.ShapeDtypeStruct((U, DH), _f32),
                   jax.ShapeDtypeStruct((G, DH), _f32)],
        interpret=_INTERPRET,
    )(global_features, w1e[3 * U:4 * U], pe['b1'].reshape(1, DH),
      pa['Wv'].reshape(U, U), pa['Wo'].reshape(U, U),
      pa['bv'].reshape(1, U), pa['bo'].reshape(1, U),
      w1n[U:2 * U], w1n[2 * U:3 * U], w1n[3 * U:4 * U],
      pn['b1'].reshape(1, DH))

    BN = 2000
    nb_n = N // BN
    P, Q = pl.pallas_call(
        _pq_body,
        grid=(nb_n,),
        in_specs=[pl.BlockSpec((BN, U), lambda i: (i, 0)),
                  pl.BlockSpec((BN, 1), lambda i: (i, 0)),
                  _full((G, DH)), _full((U, DH)), _full((U, DH))],
        out_specs=[pl.BlockSpec((BN, PW), lambda i: (i, 0)),
                   pl.BlockSpec((BN, DH), lambda i: (i, 0))],
        out_shape=[jax.ShapeDtypeStruct((N, PW), _f32),
                   jax.ShapeDtypeStruct((N, DH), _f32)],
        interpret=_INTERPRET,
    )(node_features, gidx2, gge, w1e[0:U], w1e[U:2 * U])

    src3 = edge_indices[0].reshape(32, E // (32 * 40), 40)
    dst3 = edge_indices[1].reshape(32, E // (32 * 40), 40)
    u = _make_gather(N, E, DH, PW)(P, Q, src3, dst3)

    BE = 2000
    nb_e = E // BE
    h_e, st_e = pl.pallas_call(
        _pass1e_body,
        grid=(nb_e,),
        in_specs=[pl.BlockSpec((BE, DH), lambda i: (i, 0)),
                  pl.BlockSpec((BE, U), lambda i: (i, 0)),
                  _full((U, DH))],
        out_specs=[pl.BlockSpec((BE, DH), lambda i: (i, 0)),
                   _full((DH + 1, DH))],
        out_shape=[jax.ShapeDtypeStruct((E, DH), jnp.bfloat16),
                   jax.ShapeDtypeStruct((DH + 1, DH), _f32)],
        interpret=_INTERPRET,
    )(u, edge_features, w1e[2 * U:3 * U].astype(jnp.bfloat16))

    ab_e = pl.pallas_call(
        functools.partial(_w_body, count=float(E)),
        out_shape=jax.ShapeDtypeStruct((2, DH), _f32),
        interpret=_INTERPRET,
    )(st_e, *_afn2d(pe['afn']))

    edge_out, segE, cntE = pl.pallas_call(
        _pass2e_body,
        grid=(nb_e,),
        in_specs=[pl.BlockSpec((BE, DH), lambda i: (i, 0)),
                  pl.BlockSpec((BE, 128), lambda i: (i, DH // 128)),
                  _full((2, DH)), _full((DH, U)), _full((1, U))],
        out_specs=[pl.BlockSpec((BE, U), lambda i: (i, 0)),
                   _full((GPAD, U)), _full((8, GPAD))],
        out_shape=[jax.ShapeDtypeStruct((E, U), _f32),
                   jax.ShapeDtypeStruct((GPAD, U), _f32),
                   jax.ShapeDtypeStruct((8, GPAD), _f32)],
        interpret=_INTERPRET,
    )(h_e, u, ab_e, pe['W2'].astype(jnp.bfloat16), pe['b2'].reshape(1, U))

    dst3_s = edge_indices[1].reshape(E // 128, 1, 128)
    zeros_h = jnp.zeros((624, U // 2), _f32)
    messages = _make_scatter(N, E, U)(edge_out, dst3_s, zeros_h)

    h_n, st_n = pl.pallas_call(
        _pass1n_body,
        grid=(nb_n,),
        in_specs=[pl.BlockSpec((BN, U), lambda i: (i, 0)),
                  pl.BlockSpec((BN, U), lambda i: (i, 0)),
                  pl.BlockSpec((BN, 1), lambda i: (i, 0)),
                  _full((U, DH)), _full((U, DH)), _full((G, DH))],
        out_specs=[pl.BlockSpec((BN, DH), lambda i: (i, 0)),
                   _full((DH + 1, DH))],
        out_shape=[jax.ShapeDtypeStruct((N, DH), _f32),
                   jax.ShapeDtypeStruct((DH + 1, DH), _f32)],
        interpret=_INTERPRET,
    )(node_features, messages, gidx2, w1n[0:U], k2, ggn)

    ab_n = pl.pallas_call(
        functools.partial(_w_body, count=float(N)),
        out_shape=jax.ShapeDtypeStruct((2, DH), _f32),
        interpret=_INTERPRET,
    )(st_n, *_afn2d(pn['afn']))

    node_out, segN, cntN = pl.pallas_call(
        _pass2n_body,
        grid=(nb_n,),
        in_specs=[pl.BlockSpec((BN, DH), lambda i: (i, 0)),
                  pl.BlockSpec((BN, 1), lambda i: (i, 0)),
                  _full((2, DH)), _full((DH, U)), _full((1, U))],
        out_specs=[pl.BlockSpec((BN, U), lambda i: (i, 0)),
                   _full((GPAD, U)), _full((8, GPAD))],
        out_shape=[jax.ShapeDtypeStruct((N, U), _f32),
                   jax.ShapeDtypeStruct((GPAD, U), _f32),
                   jax.ShapeDtypeStruct((8, GPAD), _f32)],
        interpret=_INTERPRET,
    )(h_n, gidx2, ab_n, pn['W2'], pn['b2'].reshape(1, U))

    global_out = pl.pallas_call(
        functools.partial(_glob_body, gcount=float(G)),
        out_shape=jax.ShapeDtypeStruct((G, U), _f32),
        interpret=_INTERPRET,
    )(global_features, segN, cntN, segE, cntE,
      w1g[0:U], w1g[U:2 * U], w1g[2 * U:3 * U], pg['b1'].reshape(1, DH),
      *_afn2d(pg['afn']), pg['W2'], pg['b2'].reshape(1, U))

    return (node_out, edge_out, global_out)
